# Initial kernel scaffold; baseline (speedup 1.0000x reference)
import jax
import jax.numpy as jnp
from jax.experimental import pallas as pl


def kernel(z, pos, batch, emb, mlp_w1, mlp_b1, mlp_w2, mlp_b2, cf1_w, cf2_w, cf2_b, il_w, il_b, hw1, hb1, hw2, hb2, ow, ob):
    raise NotImplementedError("write your pallas kernel here")


# speedup vs baseline: 2.8258x; 2.8258x over previous
import functools

import jax
import jax.numpy as jnp
import numpy as np
from jax import lax
from jax.experimental import pallas as pl
from jax.experimental.pallas import tpu as pltpu
from jax.experimental.pallas import tpu_sc as plsc

N = 4096
G = 128
K = 32
H = 64
NF = 64
NG = 50
CUT = 10.0
L = 3
LOG2 = float(np.log(2.0))

RB = 128
CT = 128
NB = N // RB
BD = 256
PI = float(np.pi)

_OFFS = np.linspace(0.0, CUT, NG).astype(np.float32)
_STEP = np.float32(_OFFS[1] - _OFFS[0])
_COEFF = np.float32(-0.5 / (_STEP * _STEP))

_NC = 2
_NS = 16
_NW = _NC * _NS
_CH = 128


def _ssp(x):
    return jax.nn.softplus(x) - LOG2


def _nbr_kernel(tlo_ref, thi_ref, pos_ref, posT_ref, bat_ref, batT_ref,
                idx_ref, cc_ref, ea_ref):
    b = pl.program_id(0)
    t0 = tlo_ref[b]
    t1 = thi_ref[b]

    xr = pos_ref[:, 0:1]
    yr = pos_ref[:, 1:2]
    zr = pos_ref[:, 2:3]
    sqr = (xr * xr + yr * yr) + zr * zr
    br = bat_ref[:]
    rowg = lax.broadcasted_iota(jnp.int32, (RB, 1), 0) + b * RB
    offs = lax.broadcasted_iota(jnp.int32, (1, NG), 1).astype(jnp.float32) * _STEP

    def score_tile(t):
        c0 = t * CT
        xc = posT_ref[0:1, pl.ds(c0, CT)]
        yc = posT_ref[1:2, pl.ds(c0, CT)]
        zc = posT_ref[2:3, pl.ds(c0, CT)]
        sqc = (xc * xc + yc * yc) + zc * zc
        bc = batT_ref[0:1, pl.ds(c0, CT)]
        colg = lax.broadcasted_iota(jnp.int32, (1, CT), 1) + c0
        dot = (xr * xc + yr * yc) + zr * zc
        d2 = sqr + sqc - 2.0 * dot
        m = (br == bc) & (rowg != colg) & (d2 <= CUT * CUT)
        s = jnp.where(m, -d2, -jnp.inf)
        return s, jnp.broadcast_to(colg, (RB, CT))

    pm = jnp.full((RB, 1), jnp.inf, jnp.float32)
    pi_ = jnp.full((RB, 1), -1, jnp.int32)
    for k in range(K):
        def tile_body(t, c):
            m_run, a_run = c
            s, colg = score_tile(t)
            elig = (s < pm) | ((s == pm) & (colg > pi_))
            s2 = jnp.where(elig, s, -jnp.inf)
            tm = jnp.max(s2, axis=1, keepdims=True)
            ta = jnp.min(jnp.where(s2 == tm, colg, N), axis=1, keepdims=True)
            better = (tm > m_run) | ((tm == m_run) & (ta < a_run))
            return (jnp.where(better, tm, m_run), jnp.where(better, ta, a_run))

        m, am = lax.fori_loop(t0, t1, tile_body,
                              (jnp.full((RB, 1), -jnp.inf, jnp.float32),
                               jnp.full((RB, 1), N, jnp.int32)))
        validb = m > -jnp.inf
        validf = validb.astype(jnp.float32)
        d = jnp.sqrt(jnp.maximum(-m, 1e-12))
        d = jnp.where(validb, d, 1.0)
        cc = (0.5 * (jnp.cos(d * PI / CUT) + 1.0)) * validf
        ea = jnp.exp(_COEFF * (d - offs) ** 2)
        idx_ref[:, k:k + 1] = jnp.where(validb, am, 0)
        cc_ref[:, k:k + 1] = cc
        ea_ref[:, k, :] = ea
        pm, pi_ = m, am


def _nbr(pos, batch, tlo, thi, interpret=False):
    posT = pos.T.reshape(3, N)
    bat2 = batch.reshape(N, 1)
    batT = batch.reshape(1, N)
    grid_spec = pltpu.PrefetchScalarGridSpec(
        num_scalar_prefetch=2,
        grid=(NB,),
        in_specs=[
            pl.BlockSpec((RB, 3), lambda b, *_: (b, 0)),
            pl.BlockSpec((3, N), lambda b, *_: (0, 0)),
            pl.BlockSpec((RB, 1), lambda b, *_: (b, 0)),
            pl.BlockSpec((1, N), lambda b, *_: (0, 0)),
        ],
        out_specs=[
            pl.BlockSpec((RB, K), lambda b, *_: (b, 0)),
            pl.BlockSpec((RB, K), lambda b, *_: (b, 0)),
            pl.BlockSpec((RB, K, NG), lambda b, *_: (b, 0, 0)),
        ],
    )
    return pl.pallas_call(
        _nbr_kernel,
        grid_spec=grid_spec,
        out_shape=[
            jax.ShapeDtypeStruct((N, K), jnp.int32),
            jax.ShapeDtypeStruct((N, K), jnp.float32),
            jax.ShapeDtypeStruct((N, K, NG), jnp.float32),
        ],
        interpret=interpret,
    )(tlo, thi, pos, posT, bat2, batT)


def _sc_gather(table, idx, D):
    B = idx.shape[0]
    bpw = B // _NW
    nch = bpw // _CH
    mesh = plsc.VectorSubcoreMesh(core_axis_name="c", subcore_axis_name="s")

    @functools.partial(
        pl.kernel,
        mesh=mesh,
        out_type=jax.ShapeDtypeStruct((B, D), jnp.float32),
        scratch_types=[
            pltpu.VMEM((_CH,), jnp.int32),
            pltpu.VMEM((_CH, D), jnp.float32),
            pltpu.SemaphoreType.DMA,
        ],
    )
    def k(table_hbm, idx_hbm, out_hbm, idx_v, rows_v, sem):
        wid = lax.axis_index("s") * _NC + lax.axis_index("c")
        base = wid * bpw

        def body(i, carry):
            off = base + i * _CH
            pltpu.sync_copy(idx_hbm.at[pl.ds(off, _CH)], idx_v)
            pltpu.async_copy(table_hbm.at[idx_v], rows_v, sem).wait()
            pltpu.sync_copy(rows_v, out_hbm.at[pl.ds(off, _CH)])
            return carry

        lax.fori_loop(0, nch, body, 0)

    return k(table, idx)


DP = 128


def _upd_kernel(ea_ref, cc_ref, xg_ref, h_ref, w1_ref, b1_ref, w2_ref,
                b2_ref, cf2w_ref, cf2b_ref, ilw_ref, ilb_ref, ho_ref):
    ea = ea_ref[:].reshape(BD * K, NG)
    t = jnp.dot(ea, w1_ref[:], preferred_element_type=jnp.float32) + b1_ref[:]
    t = _ssp(t)
    w = jnp.dot(t, w2_ref[:], preferred_element_type=jnp.float32) + b2_ref[:]
    w = w.reshape(BD, K, DP) * cc_ref[:]
    agg = jnp.sum(w * xg_ref[:], axis=1)
    t2 = jnp.dot(agg, cf2w_ref[:], preferred_element_type=jnp.float32) + cf2b_ref[:]
    t2 = _ssp(t2)
    xc = jnp.dot(t2, ilw_ref[:], preferred_element_type=jnp.float32) + ilb_ref[:]
    ho_ref[:] = h_ref[:] + xc


def _upd(ea, cc3, xg, h, w1, b1, w2p, b2p, cf2wp, cf2b, ilwp, ilbp,
         interpret=False):
    full = lambda *shape: pl.BlockSpec(shape, lambda b: tuple(0 for _ in shape))
    return pl.pallas_call(
        _upd_kernel,
        grid=(N // BD,),
        in_specs=[
            pl.BlockSpec((BD, K, NG), lambda b: (b, 0, 0)),
            pl.BlockSpec((BD, K, 1), lambda b: (b, 0, 0)),
            pl.BlockSpec((BD, K, DP), lambda b: (b, 0, 0)),
            pl.BlockSpec((BD, DP), lambda b: (b, 0)),
            full(NG, NF), full(1, NF), full(NF, DP), full(1, DP),
            full(DP, H), full(1, H), full(H, DP), full(1, DP),
        ],
        out_specs=pl.BlockSpec((BD, DP), lambda b: (b, 0)),
        out_shape=jax.ShapeDtypeStruct((N, DP), jnp.float32),
        interpret=interpret,
    )(ea, cc3, xg, h, w1, b1.reshape(1, NF), w2p, b2p.reshape(1, DP),
      cf2wp, cf2b.reshape(1, H), ilwp, ilbp.reshape(1, DP))


def _lin_kernel(x_ref, w_ref, o_ref):
    o_ref[:] = jnp.dot(x_ref[:], w_ref[:], preferred_element_type=jnp.float32)


def _lin(x, w, interpret=False):
    n, dx = x.shape
    d2 = w.shape[1]
    return pl.pallas_call(
        _lin_kernel,
        out_shape=jax.ShapeDtypeStruct((n, d2), jnp.float32),
        interpret=interpret,
    )(x, w)


def _readout_kernel(h_ref, batT_ref, hw1_ref, hb1_ref, hw2_ref, hb2_ref,
                    ow_ref, ob_ref, o_ref):
    t = jnp.dot(h_ref[:], hw1_ref[:], preferred_element_type=jnp.float32) + hb1_ref[:]
    t = _ssp(t)
    t = jnp.dot(t, hw2_ref[:], preferred_element_type=jnp.float32) + hb2_ref[:]
    gids = lax.broadcasted_iota(jnp.int32, (G, 1), 0)
    maskf = (batT_ref[:] == gids).astype(jnp.float32)
    seg = jnp.dot(maskf, t, preferred_element_type=jnp.float32)
    o_ref[:] = seg * ow_ref[0, 0] + ob_ref[0, 0]


def _readout(h, batch, hw1p, hb1, hw2, hb2, ow, ob, interpret=False):
    return pl.pallas_call(
        _readout_kernel,
        out_shape=jax.ShapeDtypeStruct((G, 1), jnp.float32),
        interpret=interpret,
    )(h, batch.reshape(1, N), hw1p, hb1.reshape(1, H // 2),
      hw2, hb2.reshape(1, 1), ow, ob.reshape(1, 1))


def kernel(z, pos, batch, emb, mlp_w1, mlp_b1, mlp_w2, mlp_b2, cf1_w, cf2_w,
           cf2_b, il_w, il_b, hw1, hb1, hw2, hb2, ow, ob):
    batch = batch.astype(jnp.int32)
    z = z.astype(jnp.int32)

    starts = jnp.searchsorted(batch, jnp.arange(G + 1, dtype=jnp.int32),
                              side="left").astype(jnp.int32)
    r0 = jnp.arange(NB, dtype=jnp.int32) * RB
    g_lo = batch[r0]
    g_hi = batch[r0 + RB - 1]
    col_lo = starts[g_lo]
    col_hi = starts[g_hi + 1]
    tlo = col_lo // CT
    thi = (col_hi + CT - 1) // CT

    idx, cc, ea = _nbr(pos, batch, tlo, thi)
    idxf = idx.reshape(N * K)
    cc3 = cc.reshape(N, K, 1)

    emb_p = jnp.pad(emb, ((0, 0), (0, DP - H)))
    h = _sc_gather(emb_p, z, DP)
    for i in range(L):
        cf1p = jnp.pad(cf1_w[i], ((0, DP - H), (0, DP - NF)))
        w2p = jnp.pad(mlp_w2[i], ((0, 0), (0, DP - NF)))
        b2p = jnp.pad(mlp_b2[i], (0, DP - NF))
        cf2wp = jnp.pad(cf2_w[i], ((0, DP - NF), (0, 0)))
        ilwp = jnp.pad(il_w[i], ((0, 0), (0, DP - H)))
        ilbp = jnp.pad(il_b[i], (0, DP - H))
        xl = _lin(h, cf1p)
        xg = _sc_gather(xl, idxf, DP).reshape(N, K, DP)
        h = _upd(ea, cc3, xg, h, mlp_w1[i], mlp_b1[i], w2p, b2p,
                 cf2wp, cf2_b[i], ilwp, ilbp)
    hw1p = jnp.pad(hw1, ((0, DP - H), (0, 0)))
    return _readout(h, batch, hw1p, hb1, hw2, hb2, ow, ob)


# speedup vs baseline: 2.8308x; 1.0017x over previous
import functools

import jax
import jax.numpy as jnp
import numpy as np
from jax import lax
from jax.experimental import pallas as pl
from jax.experimental.pallas import tpu as pltpu
from jax.experimental.pallas import tpu_sc as plsc

N = 4096
G = 128
K = 32
H = 64
NF = 64
NG = 50
CUT = 10.0
L = 3
LOG2 = float(np.log(2.0))

RB = 128
CT = 128
NB = N // RB
BD = 256
PI = float(np.pi)

_OFFS = np.linspace(0.0, CUT, NG).astype(np.float32)
_STEP = np.float32(_OFFS[1] - _OFFS[0])
_COEFF = np.float32(-0.5 / (_STEP * _STEP))

_NC = 2
_NS = 16
_NW = _NC * _NS
_CH = 128


def _ssp(x):
    return jax.nn.softplus(x) - LOG2


def _nbr_kernel(tlo_ref, thi_ref, pos_ref, posT_ref, bat_ref, batT_ref,
                idx_ref, cc_ref, ea_ref):
    b = pl.program_id(0)
    t0 = tlo_ref[b]
    t1 = thi_ref[b]

    xr = pos_ref[:, 0:1]
    yr = pos_ref[:, 1:2]
    zr = pos_ref[:, 2:3]
    sqr = (xr * xr + yr * yr) + zr * zr
    br = bat_ref[:]
    rowg = lax.broadcasted_iota(jnp.int32, (RB, 1), 0) + b * RB
    offs = lax.broadcasted_iota(jnp.int32, (1, NG), 1).astype(jnp.float32) * _STEP

    def score_tile(t):
        c0 = t * CT
        xc = posT_ref[0:1, pl.ds(c0, CT)]
        yc = posT_ref[1:2, pl.ds(c0, CT)]
        zc = posT_ref[2:3, pl.ds(c0, CT)]
        sqc = (xc * xc + yc * yc) + zc * zc
        bc = batT_ref[0:1, pl.ds(c0, CT)]
        colg = lax.broadcasted_iota(jnp.int32, (1, CT), 1) + c0
        dot = (xr * xc + yr * yc) + zr * zc
        d2 = sqr + sqc - 2.0 * dot
        m = (br == bc) & (rowg != colg) & (d2 <= CUT * CUT)
        s = jnp.where(m, -d2, -jnp.inf)
        return s, jnp.broadcast_to(colg, (RB, CT))

    pm = jnp.full((RB, 1), jnp.inf, jnp.float32)
    pi_ = jnp.full((RB, 1), -1, jnp.int32)
    for k in range(K):
        def tile_body(t, c):
            m_run, a_run = c
            s, colg = score_tile(t)
            elig = (s < pm) | ((s == pm) & (colg > pi_))
            s2 = jnp.where(elig, s, -jnp.inf)
            tm = jnp.max(s2, axis=1, keepdims=True)
            ta = jnp.min(jnp.where(s2 == tm, colg, N), axis=1, keepdims=True)
            better = (tm > m_run) | ((tm == m_run) & (ta < a_run))
            return (jnp.where(better, tm, m_run), jnp.where(better, ta, a_run))

        m, am = lax.fori_loop(t0, t1, tile_body,
                              (jnp.full((RB, 1), -jnp.inf, jnp.float32),
                               jnp.full((RB, 1), N, jnp.int32)))
        validb = m > -jnp.inf
        validf = validb.astype(jnp.float32)
        d = jnp.sqrt(jnp.maximum(-m, 1e-12))
        d = jnp.where(validb, d, 1.0)
        cc = (0.5 * (jnp.cos(d * PI / CUT) + 1.0)) * validf
        ea = jnp.exp(_COEFF * (d - offs) ** 2)
        idx_ref[:, k:k + 1] = jnp.where(validb, am, 0)
        cc_ref[:, k:k + 1] = cc
        ea_ref[:, k, :] = ea
        pm, pi_ = m, am


def _nbr(pos, batch, tlo, thi, interpret=False):
    posT = pos.T.reshape(3, N)
    bat2 = batch.reshape(N, 1)
    batT = batch.reshape(1, N)
    grid_spec = pltpu.PrefetchScalarGridSpec(
        num_scalar_prefetch=2,
        grid=(NB,),
        in_specs=[
            pl.BlockSpec((RB, 3), lambda b, *_: (b, 0)),
            pl.BlockSpec((3, N), lambda b, *_: (0, 0)),
            pl.BlockSpec((RB, 1), lambda b, *_: (b, 0)),
            pl.BlockSpec((1, N), lambda b, *_: (0, 0)),
        ],
        out_specs=[
            pl.BlockSpec((RB, K), lambda b, *_: (b, 0)),
            pl.BlockSpec((RB, K), lambda b, *_: (b, 0)),
            pl.BlockSpec((RB, K, NG), lambda b, *_: (b, 0, 0)),
        ],
    )
    return pl.pallas_call(
        _nbr_kernel,
        grid_spec=grid_spec,
        out_shape=[
            jax.ShapeDtypeStruct((N, K), jnp.int32),
            jax.ShapeDtypeStruct((N, K), jnp.float32),
            jax.ShapeDtypeStruct((N, K, NG), jnp.float32),
        ],
        interpret=interpret,
    )(tlo, thi, pos, posT, bat2, batT)


_NBUF = 4


def _sc_gather(table, idx, D):
    B = idx.shape[0]
    bpw = B // _NW
    nch = bpw // _CH
    nsup = max(nch // _NBUF, 1)
    nb = min(_NBUF, nch)
    mesh = plsc.VectorSubcoreMesh(core_axis_name="c", subcore_axis_name="s")

    @functools.partial(
        pl.kernel,
        mesh=mesh,
        out_type=jax.ShapeDtypeStruct((B, D), jnp.float32),
        scratch_types=[
            pltpu.VMEM((bpw,), jnp.int32),
            pltpu.VMEM((nb, _CH, D), jnp.float32),
        ] + [pltpu.SemaphoreType.DMA] * nb,
    )
    def k(table_hbm, idx_hbm, out_hbm, idx_v, rows_v, *sems):
        wid = lax.axis_index("s") * _NC + lax.axis_index("c")
        base = wid * bpw
        pltpu.sync_copy(idx_hbm.at[pl.ds(base, bpw)], idx_v)

        def body(i, carry):
            cps = []
            for bidx in range(nb):
                ch = i * nb + bidx
                cps.append(pltpu.async_copy(
                    table_hbm.at[idx_v.at[pl.ds(ch * _CH, _CH)]],
                    rows_v.at[bidx], sems[bidx]))
            for bidx in range(nb):
                ch = i * nb + bidx
                cps[bidx].wait()
                pltpu.sync_copy(rows_v.at[bidx],
                                out_hbm.at[pl.ds(base + ch * _CH, _CH)])
            return carry

        lax.fori_loop(0, nsup, body, 0)

    return k(table, idx)


DP = 128


def _upd_kernel(ea_ref, cc_ref, xg_ref, h_ref, w1_ref, b1_ref, w2_ref,
                b2_ref, cf2w_ref, cf2b_ref, ilw_ref, ilb_ref, ho_ref):
    ea = ea_ref[:].reshape(BD * K, NG)
    t = jnp.dot(ea, w1_ref[:], preferred_element_type=jnp.float32) + b1_ref[:]
    t = _ssp(t)
    w = jnp.dot(t, w2_ref[:], preferred_element_type=jnp.float32) + b2_ref[:]
    w = w.reshape(BD, K, DP) * cc_ref[:]
    agg = jnp.sum(w * xg_ref[:], axis=1)
    t2 = jnp.dot(agg, cf2w_ref[:], preferred_element_type=jnp.float32) + cf2b_ref[:]
    t2 = _ssp(t2)
    xc = jnp.dot(t2, ilw_ref[:], preferred_element_type=jnp.float32) + ilb_ref[:]
    ho_ref[:] = h_ref[:] + xc


def _upd(ea, cc3, xg, h, w1, b1, w2p, b2p, cf2wp, cf2b, ilwp, ilbp,
         interpret=False):
    full = lambda *shape: pl.BlockSpec(shape, lambda b: tuple(0 for _ in shape))
    return pl.pallas_call(
        _upd_kernel,
        grid=(N // BD,),
        in_specs=[
            pl.BlockSpec((BD, K, NG), lambda b: (b, 0, 0)),
            pl.BlockSpec((BD, K, 1), lambda b: (b, 0, 0)),
            pl.BlockSpec((BD, K, DP), lambda b: (b, 0, 0)),
            pl.BlockSpec((BD, DP), lambda b: (b, 0)),
            full(NG, NF), full(1, NF), full(NF, DP), full(1, DP),
            full(DP, H), full(1, H), full(H, DP), full(1, DP),
        ],
        out_specs=pl.BlockSpec((BD, DP), lambda b: (b, 0)),
        out_shape=jax.ShapeDtypeStruct((N, DP), jnp.float32),
        interpret=interpret,
    )(ea, cc3, xg, h, w1, b1.reshape(1, NF), w2p, b2p.reshape(1, DP),
      cf2wp, cf2b.reshape(1, H), ilwp, ilbp.reshape(1, DP))


def _lin_kernel(x_ref, w_ref, o_ref):
    o_ref[:] = jnp.dot(x_ref[:], w_ref[:], preferred_element_type=jnp.float32)


def _lin(x, w, interpret=False):
    n, dx = x.shape
    d2 = w.shape[1]
    return pl.pallas_call(
        _lin_kernel,
        out_shape=jax.ShapeDtypeStruct((n, d2), jnp.float32),
        interpret=interpret,
    )(x, w)


def _readout_kernel(h_ref, batT_ref, hw1_ref, hb1_ref, hw2_ref, hb2_ref,
                    ow_ref, ob_ref, o_ref):
    t = jnp.dot(h_ref[:], hw1_ref[:], preferred_element_type=jnp.float32) + hb1_ref[:]
    t = _ssp(t)
    t = jnp.dot(t, hw2_ref[:], preferred_element_type=jnp.float32) + hb2_ref[:]
    gids = lax.broadcasted_iota(jnp.int32, (G, 1), 0)
    maskf = (batT_ref[:] == gids).astype(jnp.float32)
    seg = jnp.dot(maskf, t, preferred_element_type=jnp.float32)
    o_ref[:] = seg * ow_ref[0, 0] + ob_ref[0, 0]


def _readout(h, batch, hw1p, hb1, hw2, hb2, ow, ob, interpret=False):
    return pl.pallas_call(
        _readout_kernel,
        out_shape=jax.ShapeDtypeStruct((G, 1), jnp.float32),
        interpret=interpret,
    )(h, batch.reshape(1, N), hw1p, hb1.reshape(1, H // 2),
      hw2, hb2.reshape(1, 1), ow, ob.reshape(1, 1))


def kernel(z, pos, batch, emb, mlp_w1, mlp_b1, mlp_w2, mlp_b2, cf1_w, cf2_w,
           cf2_b, il_w, il_b, hw1, hb1, hw2, hb2, ow, ob):
    batch = batch.astype(jnp.int32)
    z = z.astype(jnp.int32)

    starts = jnp.searchsorted(batch, jnp.arange(G + 1, dtype=jnp.int32),
                              side="left").astype(jnp.int32)
    r0 = jnp.arange(NB, dtype=jnp.int32) * RB
    g_lo = batch[r0]
    g_hi = batch[r0 + RB - 1]
    col_lo = starts[g_lo]
    col_hi = starts[g_hi + 1]
    tlo = col_lo // CT
    thi = (col_hi + CT - 1) // CT

    idx, cc, ea = _nbr(pos, batch, tlo, thi)
    idxf = idx.reshape(N * K)
    cc3 = cc.reshape(N, K, 1)

    emb_p = jnp.pad(emb, ((0, 0), (0, DP - H)))
    h = _sc_gather(emb_p, z, DP)
    for i in range(L):
        cf1p = jnp.pad(cf1_w[i], ((0, DP - H), (0, DP - NF)))
        w2p = jnp.pad(mlp_w2[i], ((0, 0), (0, DP - NF)))
        b2p = jnp.pad(mlp_b2[i], (0, DP - NF))
        cf2wp = jnp.pad(cf2_w[i], ((0, DP - NF), (0, 0)))
        ilwp = jnp.pad(il_w[i], ((0, 0), (0, DP - H)))
        ilbp = jnp.pad(il_b[i], (0, DP - H))
        xl = _lin(h, cf1p)
        xg = _sc_gather(xl, idxf, DP).reshape(N, K, DP)
        h = _upd(ea, cc3, xg, h, mlp_w1[i], mlp_b1[i], w2p, b2p,
                 cf2wp, cf2_b[i], ilwp, ilbp)
    hw1p = jnp.pad(hw1, ((0, DP - H), (0, 0)))
    return _readout(h, batch, hw1p, hb1, hw2, hb2, ow, ob)


# speedup vs baseline: 4.3008x; 1.5193x over previous
import functools

import jax
import jax.numpy as jnp
import numpy as np
from jax import lax
from jax.experimental import pallas as pl
from jax.experimental.pallas import tpu as pltpu
from jax.experimental.pallas import tpu_sc as plsc

N = 4096
G = 128
K = 32
H = 64
NF = 64
NG = 50
CUT = 10.0
L = 3
LOG2 = float(np.log(2.0))

RB = 128
CT = 128
NB = N // RB
BD = 256
PI = float(np.pi)

_OFFS = np.linspace(0.0, CUT, NG).astype(np.float32)
_STEP = np.float32(_OFFS[1] - _OFFS[0])
_COEFF = np.float32(-0.5 / (_STEP * _STEP))

_NC = 2
_NS = 16
_NW = _NC * _NS
_CH = 128


def _ssp(x):
    return jax.nn.softplus(x) - LOG2


def _nbr_kernel(tlo_ref, thi_ref, pos_ref, posT_ref, bat_ref, batT_ref,
                idx_ref, cc_ref, ea_ref):
    b = pl.program_id(0)
    t0 = tlo_ref[b]
    t1 = thi_ref[b]

    xr = pos_ref[:, 0:1]
    yr = pos_ref[:, 1:2]
    zr = pos_ref[:, 2:3]
    sqr = (xr * xr + yr * yr) + zr * zr
    br = bat_ref[:]
    rowg = lax.broadcasted_iota(jnp.int32, (RB, 1), 0) + b * RB
    offs = lax.broadcasted_iota(jnp.int32, (1, NG), 1).astype(jnp.float32) * _STEP

    def score_tile(t):
        c0 = t * CT
        xc = posT_ref[0:1, pl.ds(c0, CT)]
        yc = posT_ref[1:2, pl.ds(c0, CT)]
        zc = posT_ref[2:3, pl.ds(c0, CT)]
        sqc = (xc * xc + yc * yc) + zc * zc
        bc = batT_ref[0:1, pl.ds(c0, CT)]
        colg = lax.broadcasted_iota(jnp.int32, (1, CT), 1) + c0
        dot = (xr * xc + yr * yc) + zr * zc
        d2 = sqr + sqc - 2.0 * dot
        m = (br == bc) & (rowg != colg) & (d2 <= CUT * CUT)
        s = jnp.where(m, -d2, -jnp.inf)
        return s, jnp.broadcast_to(colg, (RB, CT))

    pm = jnp.full((RB, 1), jnp.inf, jnp.float32)
    pi_ = jnp.full((RB, 1), -1, jnp.int32)
    for k in range(K):
        def tile_body(t, c):
            m_run, a_run = c
            s, colg = score_tile(t)
            elig = (s < pm) | ((s == pm) & (colg > pi_))
            s2 = jnp.where(elig, s, -jnp.inf)
            tm = jnp.max(s2, axis=1, keepdims=True)
            ta = jnp.min(jnp.where(s2 == tm, colg, N), axis=1, keepdims=True)
            better = (tm > m_run) | ((tm == m_run) & (ta < a_run))
            return (jnp.where(better, tm, m_run), jnp.where(better, ta, a_run))

        m, am = lax.fori_loop(t0, t1, tile_body,
                              (jnp.full((RB, 1), -jnp.inf, jnp.float32),
                               jnp.full((RB, 1), N, jnp.int32)))
        validb = m > -jnp.inf
        validf = validb.astype(jnp.float32)
        d = jnp.sqrt(jnp.maximum(-m, 1e-12))
        d = jnp.where(validb, d, 1.0)
        cc = (0.5 * (jnp.cos(d * PI / CUT) + 1.0)) * validf
        ea = jnp.exp(_COEFF * (d - offs) ** 2)
        idx_ref[:, k:k + 1] = jnp.where(validb, am, rowg)
        cc_ref[:, k:k + 1] = cc
        ea_ref[:, k, :] = ea
        pm, pi_ = m, am


def _nbr(pos, batch, tlo, thi, interpret=False):
    posT = pos.T.reshape(3, N)
    bat2 = batch.reshape(N, 1)
    batT = batch.reshape(1, N)
    grid_spec = pltpu.PrefetchScalarGridSpec(
        num_scalar_prefetch=2,
        grid=(NB,),
        in_specs=[
            pl.BlockSpec((RB, 3), lambda b, *_: (b, 0)),
            pl.BlockSpec((3, N), lambda b, *_: (0, 0)),
            pl.BlockSpec((RB, 1), lambda b, *_: (b, 0)),
            pl.BlockSpec((1, N), lambda b, *_: (0, 0)),
        ],
        out_specs=[
            pl.BlockSpec((RB, K), lambda b, *_: (b, 0)),
            pl.BlockSpec((RB, K), lambda b, *_: (b, 0)),
            pl.BlockSpec((RB, K, NG), lambda b, *_: (b, 0, 0)),
        ],
    )
    return pl.pallas_call(
        _nbr_kernel,
        grid_spec=grid_spec,
        out_shape=[
            jax.ShapeDtypeStruct((N, K), jnp.int32),
            jax.ShapeDtypeStruct((N, K), jnp.float32),
            jax.ShapeDtypeStruct((N, K, NG), jnp.float32),
        ],
        interpret=interpret,
    )(tlo, thi, pos, posT, bat2, batT)


_NBUF = 4


def _sc_gather(table, idx, D):
    B = idx.shape[0]
    bpw = B // _NW
    nch = bpw // _CH
    nsup = max(nch // _NBUF, 1)
    nb = min(_NBUF, nch)
    mesh = plsc.VectorSubcoreMesh(core_axis_name="c", subcore_axis_name="s")

    @functools.partial(
        pl.kernel,
        mesh=mesh,
        out_type=jax.ShapeDtypeStruct((B, D), jnp.float32),
        scratch_types=[
            pltpu.VMEM((bpw,), jnp.int32),
            pltpu.VMEM((nb, _CH, D), jnp.float32),
        ] + [pltpu.SemaphoreType.DMA] * nb,
    )
    def k(table_hbm, idx_hbm, out_hbm, idx_v, rows_v, *sems):
        wid = lax.axis_index("s") * _NC + lax.axis_index("c")
        base = wid * bpw
        pltpu.sync_copy(idx_hbm.at[pl.ds(base, bpw)], idx_v)

        def body(i, carry):
            cps = []
            for bidx in range(nb):
                ch = i * nb + bidx
                cps.append(pltpu.async_copy(
                    table_hbm.at[idx_v.at[pl.ds(ch * _CH, _CH)]],
                    rows_v.at[bidx], sems[bidx]))
            for bidx in range(nb):
                ch = i * nb + bidx
                cps[bidx].wait()
                pltpu.sync_copy(rows_v.at[bidx],
                                out_hbm.at[pl.ds(base + ch * _CH, _CH)])
            return carry

        lax.fori_loop(0, nsup, body, 0)

    return k(table, idx)


DP = 128

WROWS = 512
APW = N // _NW
ECH = 256
ACH = ECH // K
NCHK = APW // ACH


def _sc_agg(xl, wf, idx):
    wfl = wf.reshape(N * K, NF)
    mesh = plsc.VectorSubcoreMesh(core_axis_name="c", subcore_axis_name="s")

    @functools.partial(
        pl.kernel,
        mesh=mesh,
        out_type=jax.ShapeDtypeStruct((N, NF), jnp.float32),
        scratch_types=[
            pltpu.VMEM((WROWS, NF), jnp.float32),
            pltpu.VMEM((ECH, NF), jnp.float32),
            pltpu.VMEM((ECH,), jnp.int32),
            pltpu.VMEM((ACH, NF), jnp.float32),
        ],
    )
    def k(xl_hbm, wf_hbm, idx_hbm, agg_hbm, win_v, wch_v, idx_v, out_v):
        wid = lax.axis_index("s") * _NC + lax.axis_index("c")
        abase = wid * APW
        ebase = abase * K
        w0 = jnp.clip(abase - (WROWS - APW) // 2, 0, N - WROWS)
        w0 = pl.multiple_of(w0, 8)
        pltpu.sync_copy(xl_hbm.at[pl.ds(w0, WROWS)], win_v)

        def chunk(c, carry):
            e0 = pl.multiple_of(ebase + c * ECH, ECH)
            pltpu.sync_copy(idx_hbm.at[pl.ds(e0, ECH)], idx_v)
            pltpu.sync_copy(wf_hbm.at[pl.ds(e0, ECH)], wch_v)

            def atom(al, carry2):
                iv0 = idx_v[pl.ds(al * K, 16)]
                iv1 = idx_v[pl.ds(al * K + 16, 16)]
                accs = [jnp.zeros((16,), jnp.float32) for _ in range(NF // 16)]
                for kk in range(K):
                    col = iv0[kk] if kk < 16 else iv1[kk - 16]
                    off = col - w0
                    erow = al * K + kk
                    for f in range(NF // 16):
                        g = win_v[off, pl.ds(f * 16, 16)]
                        wv = wch_v[erow, pl.ds(f * 16, 16)]
                        accs[f] = accs[f] + g * wv
                for f in range(NF // 16):
                    out_v[al, pl.ds(f * 16, 16)] = accs[f]
                return carry2

            lax.fori_loop(0, ACH, atom, 0)
            a0 = pl.multiple_of(abase + c * ACH, ACH)
            pltpu.sync_copy(out_v, agg_hbm.at[pl.ds(a0, ACH)])
            return carry

        lax.fori_loop(0, NCHK, chunk, 0)

    return k(xl, wfl, idx)


def _red_kernel(w_ref, xg_ref, o_ref):
    o_ref[:] = jnp.sum(w_ref[:] * xg_ref[:][:, :, :NF], axis=1)


def _red(wf, xg, interpret=False):
    return pl.pallas_call(
        _red_kernel,
        grid=(N // BD,),
        in_specs=[
            pl.BlockSpec((BD, K, NF), lambda b: (b, 0, 0)),
            pl.BlockSpec((BD, K, DP), lambda b: (b, 0, 0)),
        ],
        out_specs=pl.BlockSpec((BD, NF), lambda b: (b, 0)),
        out_shape=jax.ShapeDtypeStruct((N, NF), jnp.float32),
        interpret=interpret,
    )(wf, xg)


def _wker_kernel(ea_ref, cc_ref, w1_ref, b1_ref, w2_ref, b2_ref, w_ref):
    ea = ea_ref[:].reshape(BD * K, NG)
    t = jnp.dot(ea, w1_ref[:], preferred_element_type=jnp.float32) + b1_ref[:]
    t = _ssp(t)
    w = jnp.dot(t, w2_ref[:], preferred_element_type=jnp.float32) + b2_ref[:]
    w_ref[:] = w.reshape(BD, K, NF) * cc_ref[:]


def _wker(ea, cc3, w1, b1, w2, b2, interpret=False):
    full = lambda *shape: pl.BlockSpec(shape, lambda b: tuple(0 for _ in shape))
    return pl.pallas_call(
        _wker_kernel,
        grid=(N // BD,),
        in_specs=[
            pl.BlockSpec((BD, K, NG), lambda b: (b, 0, 0)),
            pl.BlockSpec((BD, K, 1), lambda b: (b, 0, 0)),
            full(NG, NF), full(1, NF), full(NF, NF), full(1, NF),
        ],
        out_specs=pl.BlockSpec((BD, K, NF), lambda b: (b, 0, 0)),
        out_shape=jax.ShapeDtypeStruct((N, K, NF), jnp.float32),
        interpret=interpret,
    )(ea, cc3, w1, b1.reshape(1, NF), w2, b2.reshape(1, NF))


def _upd2_kernel(agg_ref, h_ref, cf2w_ref, cf2b_ref, ilw_ref, ilb_ref, ho_ref):
    t2 = jnp.dot(agg_ref[:], cf2w_ref[:], preferred_element_type=jnp.float32) + cf2b_ref[:]
    t2 = _ssp(t2)
    xc = jnp.dot(t2, ilw_ref[:], preferred_element_type=jnp.float32) + ilb_ref[:]
    ho_ref[:] = h_ref[:][:, :H] + xc


def _upd2(agg, h, cf2w, cf2b, ilw, ilb, interpret=False):
    dh = h.shape[1]
    full = lambda *shape: pl.BlockSpec(shape, lambda b: tuple(0 for _ in shape))
    return pl.pallas_call(
        _upd2_kernel,
        grid=(N // BD,),
        in_specs=[
            pl.BlockSpec((BD, NF), lambda b: (b, 0)),
            pl.BlockSpec((BD, dh), lambda b: (b, 0)),
            full(NF, H), full(1, H), full(H, H), full(1, H),
        ],
        out_specs=pl.BlockSpec((BD, H), lambda b: (b, 0)),
        out_shape=jax.ShapeDtypeStruct((N, H), jnp.float32),
        interpret=interpret,
    )(agg, h, cf2w, cf2b.reshape(1, H), ilw, ilb.reshape(1, H))


def _lin_kernel(x_ref, w_ref, o_ref):
    o_ref[:] = jnp.dot(x_ref[:], w_ref[:], preferred_element_type=jnp.float32)


def _lin(x, w, interpret=False):
    n, dx = x.shape
    d2 = w.shape[1]
    return pl.pallas_call(
        _lin_kernel,
        out_shape=jax.ShapeDtypeStruct((n, d2), jnp.float32),
        interpret=interpret,
    )(x, w)


def _readout_kernel(h_ref, batT_ref, hw1_ref, hb1_ref, hw2_ref, hb2_ref,
                    ow_ref, ob_ref, o_ref):
    t = jnp.dot(h_ref[:], hw1_ref[:], preferred_element_type=jnp.float32) + hb1_ref[:]
    t = _ssp(t)
    t = jnp.dot(t, hw2_ref[:], preferred_element_type=jnp.float32) + hb2_ref[:]
    gids = lax.broadcasted_iota(jnp.int32, (G, 1), 0)
    maskf = (batT_ref[:] == gids).astype(jnp.float32)
    seg = jnp.dot(maskf, t, preferred_element_type=jnp.float32)
    o_ref[:] = seg * ow_ref[0, 0] + ob_ref[0, 0]


def _readout(h, batch, hw1p, hb1, hw2, hb2, ow, ob, interpret=False):
    return pl.pallas_call(
        _readout_kernel,
        out_shape=jax.ShapeDtypeStruct((G, 1), jnp.float32),
        interpret=interpret,
    )(h, batch.reshape(1, N), hw1p, hb1.reshape(1, H // 2),
      hw2, hb2.reshape(1, 1), ow, ob.reshape(1, 1))


def kernel(z, pos, batch, emb, mlp_w1, mlp_b1, mlp_w2, mlp_b2, cf1_w, cf2_w,
           cf2_b, il_w, il_b, hw1, hb1, hw2, hb2, ow, ob):
    batch = batch.astype(jnp.int32)
    z = z.astype(jnp.int32)

    starts = jnp.searchsorted(batch, jnp.arange(G + 1, dtype=jnp.int32),
                              side="left").astype(jnp.int32)
    r0 = jnp.arange(NB, dtype=jnp.int32) * RB
    g_lo = batch[r0]
    g_hi = batch[r0 + RB - 1]
    col_lo = starts[g_lo]
    col_hi = starts[g_hi + 1]
    tlo = col_lo // CT
    thi = (col_hi + CT - 1) // CT

    idx, cc, ea = _nbr(pos, batch, tlo, thi)
    idxf = idx.reshape(N * K)
    cc3 = cc.reshape(N, K, 1)

    rsub = jnp.arange(_NW, dtype=jnp.int32) * APW
    cmin = starts[batch[rsub]]
    cmax = starts[batch[rsub + APW - 1] + 1]
    w0s = jnp.clip(rsub - (WROWS - APW) // 2, 0, N - WROWS)
    fits = jnp.all((cmin >= w0s) & (cmax <= w0s + WROWS))

    emb_p = jnp.pad(emb, ((0, 0), (0, DP - H)))
    h = _sc_gather(emb_p, z, DP)

    def _agg_fast(ops):
        xl, wf, idxf = ops
        return _sc_agg(xl, wf, idxf)

    def _agg_slow(ops):
        xl, wf, idxf = ops
        xlp = jnp.pad(xl, ((0, 0), (0, DP - NF)))
        xg = _sc_gather(xlp, idxf, DP).reshape(N, K, DP)
        return _red(wf, xg)

    for i in range(L):
        cf1p = jnp.pad(cf1_w[i], ((0, h.shape[1] - H), (0, 0)))
        xl = _lin(h, cf1p)
        wf = _wker(ea, cc3, mlp_w1[i], mlp_b1[i], mlp_w2[i], mlp_b2[i])
        agg = lax.cond(fits, _agg_fast, _agg_slow, (xl, wf, idxf))
        h = _upd2(agg, h, cf2_w[i], cf2_b[i], il_w[i], il_b[i])
    return _readout(h, batch, hw1, hb1, hw2, hb2, ow, ob)


# speedup vs baseline: 6.5666x; 1.5268x over previous
import functools

import jax
import jax.numpy as jnp
import numpy as np
from jax import lax
from jax.experimental import pallas as pl
from jax.experimental.pallas import tpu as pltpu
from jax.experimental.pallas import tpu_sc as plsc

N = 4096
G = 128
K = 32
H = 64
NF = 64
NG = 50
CUT = 10.0
L = 3
LOG2 = float(np.log(2.0))

RB = 128
CT = 128
NB = N // RB
BD = 256
PI = float(np.pi)

_OFFS = np.linspace(0.0, CUT, NG).astype(np.float32)
_STEP = np.float32(_OFFS[1] - _OFFS[0])
_COEFF = np.float32(-0.5 / (_STEP * _STEP))

_NC = 2
_NS = 16
_NW = _NC * _NS
_CH = 128


def _ssp(x):
    return jax.nn.softplus(x) - LOG2


SCT = 4


def _nbr_kernel(tlo_ref, thi_ref, pos_ref, posT_ref, bat_ref, batT_ref,
                idx_ref, cc_ref, ea_ref, sc_ref):
    b = pl.program_id(0)
    t0 = tlo_ref[b]
    t1 = thi_ref[b]

    xr = pos_ref[:, 0:1]
    yr = pos_ref[:, 1:2]
    zr = pos_ref[:, 2:3]
    sqr = (xr * xr + yr * yr) + zr * zr
    br = bat_ref[:]
    rowg = lax.broadcasted_iota(jnp.int32, (RB, 1), 0) + b * RB
    offs = lax.broadcasted_iota(jnp.int32, (1, NG), 1).astype(jnp.float32) * _STEP

    def score_at(c0, tvalid):
        xc = posT_ref[0:1, pl.ds(c0, CT)]
        yc = posT_ref[1:2, pl.ds(c0, CT)]
        zc = posT_ref[2:3, pl.ds(c0, CT)]
        sqc = (xc * xc + yc * yc) + zc * zc
        bc = batT_ref[0:1, pl.ds(c0, CT)]
        colg = lax.broadcasted_iota(jnp.int32, (1, CT), 1) + c0
        dot = (xr * xc + yr * yc) + zr * zc
        d2 = sqr + sqc - 2.0 * dot
        m = (br == bc) & (rowg != colg) & (d2 <= CUT * CUT) & tvalid
        s = jnp.where(m, -d2, -jnp.inf)
        return s, jnp.broadcast_to(colg, (RB, CT))

    def _c0(j):
        return jnp.minimum((t0 + j) * CT, N - CT)

    for j in range(SCT):
        s, _ = score_at(_c0(j), (t0 + j) < t1)
        sc_ref[:, j * CT:(j + 1) * CT] = s

    pm = jnp.full((RB, 1), jnp.inf, jnp.float32)
    pi_ = jnp.full((RB, 1), -1, jnp.int32)
    for k in range(K):
        m_run = jnp.full((RB, 1), -jnp.inf, jnp.float32)
        a_run = jnp.full((RB, 1), N, jnp.int32)
        for j in range(SCT):
            s = sc_ref[:, j * CT:(j + 1) * CT]
            colg = lax.broadcasted_iota(jnp.int32, (1, CT), 1) + _c0(j)
            elig = (s < pm) | ((s == pm) & (colg > pi_))
            s2 = jnp.where(elig, s, -jnp.inf)
            tm = jnp.max(s2, axis=1, keepdims=True)
            ta = jnp.min(jnp.where(s2 == tm, colg, N), axis=1, keepdims=True)
            better = (tm > m_run) | ((tm == m_run) & (ta < a_run))
            m_run = jnp.where(better, tm, m_run)
            a_run = jnp.where(better, ta, a_run)

        def tile_body(t, c):
            m_r, a_r = c
            s, colg = score_at(t * CT, True)
            elig = (s < pm) | ((s == pm) & (colg > pi_))
            s2 = jnp.where(elig, s, -jnp.inf)
            tm = jnp.max(s2, axis=1, keepdims=True)
            ta = jnp.min(jnp.where(s2 == tm, colg, N), axis=1, keepdims=True)
            better = (tm > m_r) | ((tm == m_r) & (ta < a_r))
            return (jnp.where(better, tm, m_r), jnp.where(better, ta, a_r))

        m, am = lax.fori_loop(t0 + SCT, t1, tile_body, (m_run, a_run))
        validb = m > -jnp.inf
        validf = validb.astype(jnp.float32)
        d = jnp.sqrt(jnp.maximum(-m, 1e-12))
        d = jnp.where(validb, d, 1.0)
        cc = (0.5 * (jnp.cos(d * PI / CUT) + 1.0)) * validf
        ea = jnp.exp(_COEFF * (d - offs) ** 2)
        idx_ref[:, k:k + 1] = jnp.where(validb, am, rowg)
        cc_ref[:, k:k + 1] = cc
        ea_ref[:, k, :] = ea
        pm, pi_ = m, am


def _nbr(pos, batch, tlo, thi, interpret=False):
    posT = pos.T.reshape(3, N)
    bat2 = batch.reshape(N, 1)
    batT = batch.reshape(1, N)
    grid_spec = pltpu.PrefetchScalarGridSpec(
        num_scalar_prefetch=2,
        grid=(NB,),
        in_specs=[
            pl.BlockSpec((RB, 3), lambda b, *_: (b, 0)),
            pl.BlockSpec((3, N), lambda b, *_: (0, 0)),
            pl.BlockSpec((RB, 1), lambda b, *_: (b, 0)),
            pl.BlockSpec((1, N), lambda b, *_: (0, 0)),
        ],
        out_specs=[
            pl.BlockSpec((RB, K), lambda b, *_: (b, 0)),
            pl.BlockSpec((RB, K), lambda b, *_: (b, 0)),
            pl.BlockSpec((RB, K, NG), lambda b, *_: (b, 0, 0)),
        ],
        scratch_shapes=[pltpu.VMEM((RB, SCT * CT), jnp.float32)],
    )
    return pl.pallas_call(
        _nbr_kernel,
        grid_spec=grid_spec,
        out_shape=[
            jax.ShapeDtypeStruct((N, K), jnp.int32),
            jax.ShapeDtypeStruct((N, K), jnp.float32),
            jax.ShapeDtypeStruct((N, K, NG), jnp.float32),
        ],
        interpret=interpret,
    )(tlo, thi, pos, posT, bat2, batT)


_NBUF = 4


def _sc_gather(table, idx, D):
    B = idx.shape[0]
    bpw = B // _NW
    nch = bpw // _CH
    nsup = max(nch // _NBUF, 1)
    nb = min(_NBUF, nch)
    mesh = plsc.VectorSubcoreMesh(core_axis_name="c", subcore_axis_name="s")

    @functools.partial(
        pl.kernel,
        mesh=mesh,
        out_type=jax.ShapeDtypeStruct((B, D), jnp.float32),
        scratch_types=[
            pltpu.VMEM((bpw,), jnp.int32),
            pltpu.VMEM((nb, _CH, D), jnp.float32),
        ] + [pltpu.SemaphoreType.DMA] * nb,
    )
    def k(table_hbm, idx_hbm, out_hbm, idx_v, rows_v, *sems):
        wid = lax.axis_index("s") * _NC + lax.axis_index("c")
        base = wid * bpw
        pltpu.sync_copy(idx_hbm.at[pl.ds(base, bpw)], idx_v)

        def body(i, carry):
            cps = []
            for bidx in range(nb):
                ch = i * nb + bidx
                cps.append(pltpu.async_copy(
                    table_hbm.at[idx_v.at[pl.ds(ch * _CH, _CH)]],
                    rows_v.at[bidx], sems[bidx]))
            for bidx in range(nb):
                ch = i * nb + bidx
                cps[bidx].wait()
                pltpu.sync_copy(rows_v.at[bidx],
                                out_hbm.at[pl.ds(base + ch * _CH, _CH)])
            return carry

        lax.fori_loop(0, nsup, body, 0)

    return k(table, idx)


DP = 128

WROWS = 512
APW = N // _NW
ECH = 256
ACH = ECH // K
NCHK = APW // ACH


def _sc_agg(xl, wf, idx):
    wfl = wf.reshape(N * K, NF)
    mesh = plsc.VectorSubcoreMesh(core_axis_name="c", subcore_axis_name="s")

    @functools.partial(
        pl.kernel,
        mesh=mesh,
        out_type=jax.ShapeDtypeStruct((N, NF), jnp.float32),
        scratch_types=[
            pltpu.VMEM((WROWS, NF), jnp.float32),
            pltpu.VMEM((ECH, NF), jnp.float32),
            pltpu.VMEM((ECH,), jnp.int32),
            pltpu.VMEM((ACH, NF), jnp.float32),
        ],
    )
    def k(xl_hbm, wf_hbm, idx_hbm, agg_hbm, win_v, wch_v, idx_v, out_v):
        wid = lax.axis_index("s") * _NC + lax.axis_index("c")
        abase = wid * APW
        ebase = abase * K
        w0 = jnp.clip(abase - (WROWS - APW) // 2, 0, N - WROWS)
        w0 = pl.multiple_of(w0, 8)
        pltpu.sync_copy(xl_hbm.at[pl.ds(w0, WROWS)], win_v)

        def chunk(c, carry):
            e0 = pl.multiple_of(ebase + c * ECH, ECH)
            pltpu.sync_copy(idx_hbm.at[pl.ds(e0, ECH)], idx_v)
            pltpu.sync_copy(wf_hbm.at[pl.ds(e0, ECH)], wch_v)

            def atom(al, carry2):
                iv0 = idx_v[pl.ds(al * K, 16)]
                iv1 = idx_v[pl.ds(al * K + 16, 16)]
                accs = [jnp.zeros((16,), jnp.float32) for _ in range(NF // 16)]
                for kk in range(K):
                    col = iv0[kk] if kk < 16 else iv1[kk - 16]
                    off = col - w0
                    erow = al * K + kk
                    for f in range(NF // 16):
                        g = win_v[off, pl.ds(f * 16, 16)]
                        wv = wch_v[erow, pl.ds(f * 16, 16)]
                        accs[f] = accs[f] + g * wv
                for f in range(NF // 16):
                    out_v[al, pl.ds(f * 16, 16)] = accs[f]
                return carry2

            lax.fori_loop(0, ACH, atom, 0)
            a0 = pl.multiple_of(abase + c * ACH, ACH)
            pltpu.sync_copy(out_v, agg_hbm.at[pl.ds(a0, ACH)])
            return carry

        lax.fori_loop(0, NCHK, chunk, 0)

    return k(xl, wfl, idx)


def _red_kernel(w_ref, xg_ref, o_ref):
    o_ref[:] = jnp.sum(w_ref[:] * xg_ref[:][:, :, :NF], axis=1)


def _red(wf, xg, interpret=False):
    return pl.pallas_call(
        _red_kernel,
        grid=(N // BD,),
        in_specs=[
            pl.BlockSpec((BD, K, NF), lambda b: (b, 0, 0)),
            pl.BlockSpec((BD, K, DP), lambda b: (b, 0, 0)),
        ],
        out_specs=pl.BlockSpec((BD, NF), lambda b: (b, 0)),
        out_shape=jax.ShapeDtypeStruct((N, NF), jnp.float32),
        interpret=interpret,
    )(wf, xg)


def _wker_kernel(ea_ref, cc_ref, w1_ref, b1_ref, w2_ref, b2_ref, w_ref):
    ea = ea_ref[:].reshape(BD * K, NG)
    t = jnp.dot(ea, w1_ref[:], preferred_element_type=jnp.float32) + b1_ref[:]
    t = _ssp(t)
    w = jnp.dot(t, w2_ref[:], preferred_element_type=jnp.float32) + b2_ref[:]
    w_ref[:] = w.reshape(BD, K, NF) * cc_ref[:]


def _wker(ea, cc3, w1, b1, w2, b2, interpret=False):
    full = lambda *shape: pl.BlockSpec(shape, lambda b: tuple(0 for _ in shape))
    return pl.pallas_call(
        _wker_kernel,
        grid=(N // BD,),
        in_specs=[
            pl.BlockSpec((BD, K, NG), lambda b: (b, 0, 0)),
            pl.BlockSpec((BD, K, 1), lambda b: (b, 0, 0)),
            full(NG, NF), full(1, NF), full(NF, NF), full(1, NF),
        ],
        out_specs=pl.BlockSpec((BD, K, NF), lambda b: (b, 0, 0)),
        out_shape=jax.ShapeDtypeStruct((N, K, NF), jnp.float32),
        interpret=interpret,
    )(ea, cc3, w1, b1.reshape(1, NF), w2, b2.reshape(1, NF))


def _upd2_kernel(agg_ref, h_ref, cf2w_ref, cf2b_ref, ilw_ref, ilb_ref, ho_ref):
    t2 = jnp.dot(agg_ref[:], cf2w_ref[:], preferred_element_type=jnp.float32) + cf2b_ref[:]
    t2 = _ssp(t2)
    xc = jnp.dot(t2, ilw_ref[:], preferred_element_type=jnp.float32) + ilb_ref[:]
    ho_ref[:] = h_ref[:][:, :H] + xc


def _upd2(agg, h, cf2w, cf2b, ilw, ilb, interpret=False):
    dh = h.shape[1]
    full = lambda *shape: pl.BlockSpec(shape, lambda b: tuple(0 for _ in shape))
    return pl.pallas_call(
        _upd2_kernel,
        grid=(N // BD,),
        in_specs=[
            pl.BlockSpec((BD, NF), lambda b: (b, 0)),
            pl.BlockSpec((BD, dh), lambda b: (b, 0)),
            full(NF, H), full(1, H), full(H, H), full(1, H),
        ],
        out_specs=pl.BlockSpec((BD, H), lambda b: (b, 0)),
        out_shape=jax.ShapeDtypeStruct((N, H), jnp.float32),
        interpret=interpret,
    )(agg, h, cf2w, cf2b.reshape(1, H), ilw, ilb.reshape(1, H))


def _lin_kernel(x_ref, w_ref, o_ref):
    o_ref[:] = jnp.dot(x_ref[:], w_ref[:], preferred_element_type=jnp.float32)


def _lin(x, w, interpret=False):
    n, dx = x.shape
    d2 = w.shape[1]
    return pl.pallas_call(
        _lin_kernel,
        out_shape=jax.ShapeDtypeStruct((n, d2), jnp.float32),
        interpret=interpret,
    )(x, w)


def _readout_kernel(h_ref, batT_ref, hw1_ref, hb1_ref, hw2_ref, hb2_ref,
                    ow_ref, ob_ref, o_ref):
    t = jnp.dot(h_ref[:], hw1_ref[:], preferred_element_type=jnp.float32) + hb1_ref[:]
    t = _ssp(t)
    t = jnp.dot(t, hw2_ref[:], preferred_element_type=jnp.float32) + hb2_ref[:]
    gids = lax.broadcasted_iota(jnp.int32, (G, 1), 0)
    maskf = (batT_ref[:] == gids).astype(jnp.float32)
    seg = jnp.dot(maskf, t, preferred_element_type=jnp.float32)
    o_ref[:] = seg * ow_ref[0, 0] + ob_ref[0, 0]


def _readout(h, batch, hw1p, hb1, hw2, hb2, ow, ob, interpret=False):
    return pl.pallas_call(
        _readout_kernel,
        out_shape=jax.ShapeDtypeStruct((G, 1), jnp.float32),
        interpret=interpret,
    )(h, batch.reshape(1, N), hw1p, hb1.reshape(1, H // 2),
      hw2, hb2.reshape(1, 1), ow, ob.reshape(1, 1))


def kernel(z, pos, batch, emb, mlp_w1, mlp_b1, mlp_w2, mlp_b2, cf1_w, cf2_w,
           cf2_b, il_w, il_b, hw1, hb1, hw2, hb2, ow, ob):
    batch = batch.astype(jnp.int32)
    z = z.astype(jnp.int32)

    starts = jnp.searchsorted(batch, jnp.arange(G + 1, dtype=jnp.int32),
                              side="left").astype(jnp.int32)
    r0 = jnp.arange(NB, dtype=jnp.int32) * RB
    g_lo = batch[r0]
    g_hi = batch[r0 + RB - 1]
    col_lo = starts[g_lo]
    col_hi = starts[g_hi + 1]
    tlo = col_lo // CT
    thi = (col_hi + CT - 1) // CT

    idx, cc, ea = _nbr(pos, batch, tlo, thi)
    idxf = idx.reshape(N * K)
    cc3 = cc.reshape(N, K, 1)

    rsub = jnp.arange(_NW, dtype=jnp.int32) * APW
    cmin = starts[batch[rsub]]
    cmax = starts[batch[rsub + APW - 1] + 1]
    w0s = jnp.clip(rsub - (WROWS - APW) // 2, 0, N - WROWS)
    fits = jnp.all((cmin >= w0s) & (cmax <= w0s + WROWS))

    emb_p = jnp.pad(emb, ((0, 0), (0, DP - H)))
    h = _sc_gather(emb_p, z, DP)

    def _agg_fast(ops):
        xl, wf, idxf = ops
        return _sc_agg(xl, wf, idxf)

    def _agg_slow(ops):
        xl, wf, idxf = ops
        xlp = jnp.pad(xl, ((0, 0), (0, DP - NF)))
        xg = _sc_gather(xlp, idxf, DP).reshape(N, K, DP)
        return _red(wf, xg)

    for i in range(L):
        cf1p = jnp.pad(cf1_w[i], ((0, h.shape[1] - H), (0, 0)))
        xl = _lin(h, cf1p)
        wf = _wker(ea, cc3, mlp_w1[i], mlp_b1[i], mlp_w2[i], mlp_b2[i])
        agg = lax.cond(fits, _agg_fast, _agg_slow, (xl, wf, idxf))
        h = _upd2(agg, h, cf2_w[i], cf2_b[i], il_w[i], il_b[i])
    return _readout(h, batch, hw1, hb1, hw2, hb2, ow, ob)


# speedup vs baseline: 9.3748x; 1.4277x over previous
import functools

import jax
import jax.numpy as jnp
import numpy as np
from jax import lax
from jax.experimental import pallas as pl
from jax.experimental.pallas import tpu as pltpu
from jax.experimental.pallas import tpu_sc as plsc

N = 4096
G = 128
K = 32
H = 64
NF = 64
NG = 50
CUT = 10.0
L = 3
LOG2 = float(np.log(2.0))

RB = 128
CT = 128
NB = N // RB
BD = 256
PI = float(np.pi)

_OFFS = np.linspace(0.0, CUT, NG).astype(np.float32)
_STEP = np.float32(_OFFS[1] - _OFFS[0])
_COEFF = np.float32(-0.5 / (_STEP * _STEP))

_NC = 2
_NS = 16
_NW = _NC * _NS
_CH = 128


def _ssp(x):
    return jax.nn.softplus(x) - LOG2


SCT = 3


def _nbr_kernel(tlo_ref, thi_ref, pos_ref, posT_ref, bat_ref, batT_ref,
                idx_ref, cc_ref, d_ref, sc_ref, sm_ref):
    b = pl.program_id(0)
    t0 = tlo_ref[b]
    t1 = thi_ref[b]

    xr = pos_ref[:, 0:1]
    yr = pos_ref[:, 1:2]
    zr = pos_ref[:, 2:3]
    sqr = (xr * xr + yr * yr) + zr * zr
    br = bat_ref[:]
    rowg = lax.broadcasted_iota(jnp.int32, (RB, 1), 0) + b * RB
    rowgf = rowg.astype(jnp.float32)
    BIGF = jnp.float32(N)

    def score_at(c0, tvalid):
        xc = posT_ref[0:1, pl.ds(c0, CT)]
        yc = posT_ref[1:2, pl.ds(c0, CT)]
        zc = posT_ref[2:3, pl.ds(c0, CT)]
        sqc = (xc * xc + yc * yc) + zc * zc
        bc = batT_ref[0:1, pl.ds(c0, CT)]
        colg = lax.broadcasted_iota(jnp.int32, (1, CT), 1) + c0
        dot = (xr * xc + yr * yc) + zr * zc
        d2 = sqr + sqc - 2.0 * dot
        m = (br == bc) & (rowg != colg) & (d2 <= CUT * CUT) & tvalid
        s = jnp.where(m, -d2, -jnp.inf)
        colgf = (lax.broadcasted_iota(jnp.int32, (1, CT), 1).astype(jnp.float32)
                 + c0.astype(jnp.float32))
        return s, jnp.broadcast_to(colgf, (RB, CT))

    def _c0(j):
        return jnp.minimum((t0 + j) * CT, N - CT)

    for j in range(SCT):
        s, _ = score_at(_c0(j), (t0 + j) < t1)
        sc_ref[:, j * CT:(j + 1) * CT] = s

    def peel_tile(s, colgf, pm, pif, m_run, a_run):
        elig = (s < pm) | ((s == pm) & (colgf > pif))
        s2 = jnp.where(elig, s, -jnp.inf)
        tm = jnp.max(s2, axis=1, keepdims=True)
        ta = jnp.min(jnp.where(s2 == tm, colgf, BIGF), axis=1, keepdims=True)
        better = (tm > m_run) | ((tm == m_run) & (ta < a_run))
        return jnp.where(better, tm, m_run), jnp.where(better, ta, a_run)

    pm = jnp.full((RB, 1), jnp.inf, jnp.float32)
    pif = jnp.full((RB, 1), -1.0, jnp.float32)
    for k in range(K):
        m_run = jnp.full((RB, 1), -jnp.inf, jnp.float32)
        a_run = jnp.full((RB, 1), BIGF, jnp.float32)
        for j in range(SCT):
            s = sc_ref[:, j * CT:(j + 1) * CT]
            colgf = (lax.broadcasted_iota(jnp.int32, (1, CT), 1).astype(jnp.float32)
                     + _c0(j).astype(jnp.float32))
            m_run, a_run = peel_tile(s, jnp.broadcast_to(colgf, (RB, CT)),
                                     pm, pif, m_run, a_run)

        def tile_body(t, c):
            s, colgf = score_at(t * CT, True)
            return peel_tile(s, colgf, pm, pif, c[0], c[1])

        m, am = lax.fori_loop(t0 + SCT, t1, tile_body, (m_run, a_run))
        validb = m > -jnp.inf
        idx_ref[:, k:k + 1] = jnp.where(validb, am, rowgf).astype(jnp.int32)
        sm_ref[:, k:k + 1] = m
        pm, pif = m, am

    mall = sm_ref[:]
    validb = mall > -jnp.inf
    d = jnp.sqrt(jnp.maximum(-mall, 1e-12))
    d = jnp.where(validb, d, 1.0)
    cc_ref[:] = (0.5 * (jnp.cos(d * PI / CUT) + 1.0)) * validb.astype(jnp.float32)
    d_ref[:] = d


def _nbr(pos, batch, tlo, thi, interpret=False):
    posT = pos.T.reshape(3, N)
    bat2 = batch.reshape(N, 1)
    batT = batch.reshape(1, N)
    grid_spec = pltpu.PrefetchScalarGridSpec(
        num_scalar_prefetch=2,
        grid=(NB,),
        in_specs=[
            pl.BlockSpec((RB, 3), lambda b, *_: (b, 0)),
            pl.BlockSpec((3, N), lambda b, *_: (0, 0)),
            pl.BlockSpec((RB, 1), lambda b, *_: (b, 0)),
            pl.BlockSpec((1, N), lambda b, *_: (0, 0)),
        ],
        out_specs=[
            pl.BlockSpec((RB, K), lambda b, *_: (b, 0)),
            pl.BlockSpec((RB, K), lambda b, *_: (b, 0)),
            pl.BlockSpec((RB, K), lambda b, *_: (b, 0)),
        ],
        scratch_shapes=[pltpu.VMEM((RB, SCT * CT), jnp.float32),
                        pltpu.VMEM((RB, K), jnp.float32)],
    )
    return pl.pallas_call(
        _nbr_kernel,
        grid_spec=grid_spec,
        out_shape=[
            jax.ShapeDtypeStruct((N, K), jnp.int32),
            jax.ShapeDtypeStruct((N, K), jnp.float32),
            jax.ShapeDtypeStruct((N, K), jnp.float32),
        ],
        interpret=interpret,
    )(tlo, thi, pos, posT, bat2, batT)


_NBUF = 4


def _sc_gather(table, idx, D):
    B = idx.shape[0]
    bpw = B // _NW
    nch = bpw // _CH
    nsup = max(nch // _NBUF, 1)
    nb = min(_NBUF, nch)
    mesh = plsc.VectorSubcoreMesh(core_axis_name="c", subcore_axis_name="s")

    @functools.partial(
        pl.kernel,
        mesh=mesh,
        out_type=jax.ShapeDtypeStruct((B, D), jnp.float32),
        scratch_types=[
            pltpu.VMEM((bpw,), jnp.int32),
            pltpu.VMEM((nb, _CH, D), jnp.float32),
        ] + [pltpu.SemaphoreType.DMA] * nb,
    )
    def k(table_hbm, idx_hbm, out_hbm, idx_v, rows_v, *sems):
        wid = lax.axis_index("s") * _NC + lax.axis_index("c")
        base = wid * bpw
        pltpu.sync_copy(idx_hbm.at[pl.ds(base, bpw)], idx_v)

        def body(i, carry):
            cps = []
            for bidx in range(nb):
                ch = i * nb + bidx
                cps.append(pltpu.async_copy(
                    table_hbm.at[idx_v.at[pl.ds(ch * _CH, _CH)]],
                    rows_v.at[bidx], sems[bidx]))
            for bidx in range(nb):
                ch = i * nb + bidx
                cps[bidx].wait()
                pltpu.sync_copy(rows_v.at[bidx],
                                out_hbm.at[pl.ds(base + ch * _CH, _CH)])
            return carry

        lax.fori_loop(0, nsup, body, 0)

    return k(table, idx)


DP = 128

WROWS = 512
APW = N // _NW
ECH = 256
ACH = ECH // K
NCHK = APW // ACH


def _sc_agg(xl, wf, idx):
    wfl = wf.reshape(N * K, NF)
    mesh = plsc.VectorSubcoreMesh(core_axis_name="c", subcore_axis_name="s")

    @functools.partial(
        pl.kernel,
        mesh=mesh,
        out_type=jax.ShapeDtypeStruct((N, NF), jnp.float32),
        scratch_types=[
            pltpu.VMEM((WROWS, NF), jnp.float32),
            pltpu.VMEM((ECH, NF), jnp.float32),
            pltpu.VMEM((ECH,), jnp.int32),
            pltpu.VMEM((ACH, NF), jnp.float32),
        ],
    )
    def k(xl_hbm, wf_hbm, idx_hbm, agg_hbm, win_v, wch_v, idx_v, out_v):
        wid = lax.axis_index("s") * _NC + lax.axis_index("c")
        abase = wid * APW
        ebase = abase * K
        w0 = jnp.clip(abase - (WROWS - APW) // 2, 0, N - WROWS)
        w0 = pl.multiple_of(w0, 8)
        pltpu.sync_copy(xl_hbm.at[pl.ds(w0, WROWS)], win_v)

        def chunk(c, carry):
            e0 = pl.multiple_of(ebase + c * ECH, ECH)
            pltpu.sync_copy(idx_hbm.at[pl.ds(e0, ECH)], idx_v)
            pltpu.sync_copy(wf_hbm.at[pl.ds(e0, ECH)], wch_v)

            def atom(al, carry2):
                iv0 = idx_v[pl.ds(al * K, 16)]
                iv1 = idx_v[pl.ds(al * K + 16, 16)]
                accs = [jnp.zeros((16,), jnp.float32) for _ in range(NF // 16)]
                for kk in range(K):
                    col = iv0[kk] if kk < 16 else iv1[kk - 16]
                    off = col - w0
                    erow = al * K + kk
                    for f in range(NF // 16):
                        g = win_v[off, pl.ds(f * 16, 16)]
                        wv = wch_v[erow, pl.ds(f * 16, 16)]
                        accs[f] = accs[f] + g * wv
                for f in range(NF // 16):
                    out_v[al, pl.ds(f * 16, 16)] = accs[f]
                return carry2

            lax.fori_loop(0, ACH, atom, 0)
            a0 = pl.multiple_of(abase + c * ACH, ACH)
            pltpu.sync_copy(out_v, agg_hbm.at[pl.ds(a0, ACH)])
            return carry

        lax.fori_loop(0, NCHK, chunk, 0)

    return k(xl, wfl, idx)


def _red_kernel(w_ref, xg_ref, o_ref):
    o_ref[:] = jnp.sum(w_ref[:] * xg_ref[:][:, :, :NF], axis=1)


def _red(wf, xg, interpret=False):
    return pl.pallas_call(
        _red_kernel,
        grid=(N // BD,),
        in_specs=[
            pl.BlockSpec((BD, K, NF), lambda b: (b, 0, 0)),
            pl.BlockSpec((BD, K, DP), lambda b: (b, 0, 0)),
        ],
        out_specs=pl.BlockSpec((BD, NF), lambda b: (b, 0)),
        out_shape=jax.ShapeDtypeStruct((N, NF), jnp.float32),
        interpret=interpret,
    )(wf, xg)


def _wker_kernel(d_ref, cc_ref, w1_ref, b1_ref, w2_ref, b2_ref, w_ref):
    offs = lax.broadcasted_iota(jnp.int32, (1, NG), 1).astype(jnp.float32) * _STEP
    d = d_ref[:]
    ea = jnp.exp(_COEFF * (d - offs) ** 2)
    t = jnp.dot(ea, w1_ref[:], preferred_element_type=jnp.float32) + b1_ref[:]
    t = _ssp(t)
    w = jnp.dot(t, w2_ref[:], preferred_element_type=jnp.float32) + b2_ref[:]
    w_ref[:] = w.reshape(BD, K, NF) * cc_ref[:]


def _wker(d, cc3, w1, b1, w2, b2, interpret=False):
    full = lambda *shape: pl.BlockSpec(shape, lambda b: tuple(0 for _ in shape))
    return pl.pallas_call(
        _wker_kernel,
        grid=(N // BD,),
        in_specs=[
            pl.BlockSpec((BD * K, 1), lambda b: (b, 0)),
            pl.BlockSpec((BD, K, 1), lambda b: (b, 0, 0)),
            full(NG, NF), full(1, NF), full(NF, NF), full(1, NF),
        ],
        out_specs=pl.BlockSpec((BD, K, NF), lambda b: (b, 0, 0)),
        out_shape=jax.ShapeDtypeStruct((N, K, NF), jnp.float32),
        interpret=interpret,
    )(d, cc3, w1, b1.reshape(1, NF), w2, b2.reshape(1, NF))


def _upd2_kernel(agg_ref, h_ref, cf2w_ref, cf2b_ref, ilw_ref, ilb_ref, ho_ref):
    t2 = jnp.dot(agg_ref[:], cf2w_ref[:], preferred_element_type=jnp.float32) + cf2b_ref[:]
    t2 = _ssp(t2)
    xc = jnp.dot(t2, ilw_ref[:], preferred_element_type=jnp.float32) + ilb_ref[:]
    ho_ref[:] = h_ref[:][:, :H] + xc


def _upd2(agg, h, cf2w, cf2b, ilw, ilb, interpret=False):
    dh = h.shape[1]
    full = lambda *shape: pl.BlockSpec(shape, lambda b: tuple(0 for _ in shape))
    return pl.pallas_call(
        _upd2_kernel,
        grid=(N // BD,),
        in_specs=[
            pl.BlockSpec((BD, NF), lambda b: (b, 0)),
            pl.BlockSpec((BD, dh), lambda b: (b, 0)),
            full(NF, H), full(1, H), full(H, H), full(1, H),
        ],
        out_specs=pl.BlockSpec((BD, H), lambda b: (b, 0)),
        out_shape=jax.ShapeDtypeStruct((N, H), jnp.float32),
        interpret=interpret,
    )(agg, h, cf2w, cf2b.reshape(1, H), ilw, ilb.reshape(1, H))


def _lin_kernel(x_ref, w_ref, o_ref):
    o_ref[:] = jnp.dot(x_ref[:], w_ref[:], preferred_element_type=jnp.float32)


def _lin(x, w, interpret=False):
    n, dx = x.shape
    d2 = w.shape[1]
    return pl.pallas_call(
        _lin_kernel,
        out_shape=jax.ShapeDtypeStruct((n, d2), jnp.float32),
        interpret=interpret,
    )(x, w)


def _readout_kernel(h_ref, batT_ref, hw1_ref, hb1_ref, hw2_ref, hb2_ref,
                    ow_ref, ob_ref, o_ref):
    t = jnp.dot(h_ref[:], hw1_ref[:], preferred_element_type=jnp.float32) + hb1_ref[:]
    t = _ssp(t)
    t = jnp.dot(t, hw2_ref[:], preferred_element_type=jnp.float32) + hb2_ref[:]
    gids = lax.broadcasted_iota(jnp.int32, (G, 1), 0)
    maskf = (batT_ref[:] == gids).astype(jnp.float32)
    seg = jnp.dot(maskf, t, preferred_element_type=jnp.float32)
    o_ref[:] = seg * ow_ref[0, 0] + ob_ref[0, 0]


def _readout(h, batch, hw1p, hb1, hw2, hb2, ow, ob, interpret=False):
    return pl.pallas_call(
        _readout_kernel,
        out_shape=jax.ShapeDtypeStruct((G, 1), jnp.float32),
        interpret=interpret,
    )(h, batch.reshape(1, N), hw1p, hb1.reshape(1, H // 2),
      hw2, hb2.reshape(1, 1), ow, ob.reshape(1, 1))


def kernel(z, pos, batch, emb, mlp_w1, mlp_b1, mlp_w2, mlp_b2, cf1_w, cf2_w,
           cf2_b, il_w, il_b, hw1, hb1, hw2, hb2, ow, ob):
    batch = batch.astype(jnp.int32)
    z = z.astype(jnp.int32)

    starts = jnp.searchsorted(batch, jnp.arange(G + 1, dtype=jnp.int32),
                              side="left").astype(jnp.int32)
    r0 = jnp.arange(NB, dtype=jnp.int32) * RB
    g_lo = batch[r0]
    g_hi = batch[r0 + RB - 1]
    col_lo = starts[g_lo]
    col_hi = starts[g_hi + 1]
    tlo = col_lo // CT
    thi = (col_hi + CT - 1) // CT

    idx, cc, dmat = _nbr(pos, batch, tlo, thi)
    idxf = idx.reshape(N * K)
    cc3 = cc.reshape(N, K, 1)
    dflat = dmat.reshape(N * K, 1)

    rsub = jnp.arange(_NW, dtype=jnp.int32) * APW
    cmin = starts[batch[rsub]]
    cmax = starts[batch[rsub + APW - 1] + 1]
    w0s = jnp.clip(rsub - (WROWS - APW) // 2, 0, N - WROWS)
    fits = jnp.all((cmin >= w0s) & (cmax <= w0s + WROWS))

    emb_p = jnp.pad(emb, ((0, 0), (0, DP - H)))
    h = _sc_gather(emb_p, z, DP)

    def _agg_fast(ops):
        xl, wf, idxf = ops
        return _sc_agg(xl, wf, idxf)

    def _agg_slow(ops):
        xl, wf, idxf = ops
        xlp = jnp.pad(xl, ((0, 0), (0, DP - NF)))
        xg = _sc_gather(xlp, idxf, DP).reshape(N, K, DP)
        return _red(wf, xg)

    for i in range(L):
        cf1p = jnp.pad(cf1_w[i], ((0, h.shape[1] - H), (0, 0)))
        xl = _lin(h, cf1p)
        wf = _wker(dflat, cc3, mlp_w1[i], mlp_b1[i], mlp_w2[i], mlp_b2[i])
        agg = lax.cond(fits, _agg_fast, _agg_slow, (xl, wf, idxf))
        h = _upd2(agg, h, cf2_w[i], cf2_b[i], il_w[i], il_b[i])
    return _readout(h, batch, hw1, hb1, hw2, hb2, ow, ob)


# speedup vs baseline: 9.4987x; 1.0132x over previous
import functools

import jax
import jax.numpy as jnp
import numpy as np
from jax import lax
from jax.experimental import pallas as pl
from jax.experimental.pallas import tpu as pltpu
from jax.experimental.pallas import tpu_sc as plsc

N = 4096
G = 128
K = 32
H = 64
NF = 64
NG = 50
CUT = 10.0
L = 3
LOG2 = float(np.log(2.0))

RB = 128
CT = 128
NB = N // RB
BD = 512
PI = float(np.pi)

_OFFS = np.linspace(0.0, CUT, NG).astype(np.float32)
_STEP = np.float32(_OFFS[1] - _OFFS[0])
_COEFF = np.float32(-0.5 / (_STEP * _STEP))

_NC = 2
_NS = 16
_NW = _NC * _NS
_CH = 128


def _ssp(x):
    return jax.nn.softplus(x) - LOG2


SCT = 3


def _nbr_kernel(tlo_ref, thi_ref, pos_ref, posT_ref, bat_ref, batT_ref,
                idx_ref, cc_ref, d_ref, sc_ref, sm_ref):
    b = pl.program_id(0)
    t0 = tlo_ref[b]
    t1 = thi_ref[b]

    xr = pos_ref[:, 0:1]
    yr = pos_ref[:, 1:2]
    zr = pos_ref[:, 2:3]
    sqr = (xr * xr + yr * yr) + zr * zr
    br = bat_ref[:]
    rowg = lax.broadcasted_iota(jnp.int32, (RB, 1), 0) + b * RB
    rowgf = rowg.astype(jnp.float32)
    BIGF = jnp.float32(N)

    def score_at(c0, tvalid):
        xc = posT_ref[0:1, pl.ds(c0, CT)]
        yc = posT_ref[1:2, pl.ds(c0, CT)]
        zc = posT_ref[2:3, pl.ds(c0, CT)]
        sqc = (xc * xc + yc * yc) + zc * zc
        bc = batT_ref[0:1, pl.ds(c0, CT)]
        colg = lax.broadcasted_iota(jnp.int32, (1, CT), 1) + c0
        dot = (xr * xc + yr * yc) + zr * zc
        d2 = sqr + sqc - 2.0 * dot
        m = (br == bc) & (rowg != colg) & (d2 <= CUT * CUT) & tvalid
        s = jnp.where(m, -d2, -jnp.inf)
        colgf = (lax.broadcasted_iota(jnp.int32, (1, CT), 1).astype(jnp.float32)
                 + c0.astype(jnp.float32))
        return s, jnp.broadcast_to(colgf, (RB, CT))

    def _c0(j):
        return jnp.minimum((t0 + j) * CT, N - CT)

    for j in range(SCT):
        s, _ = score_at(_c0(j), (t0 + j) < t1)
        sc_ref[:, j * CT:(j + 1) * CT] = s

    def peel_tile(s, colgf, pm, pif, m_run, a_run):
        elig = (s < pm) | ((s == pm) & (colgf > pif))
        s2 = jnp.where(elig, s, -jnp.inf)
        tm = jnp.max(s2, axis=1, keepdims=True)
        ta = jnp.min(jnp.where(s2 == tm, colgf, BIGF), axis=1, keepdims=True)
        better = (tm > m_run) | ((tm == m_run) & (ta < a_run))
        return jnp.where(better, tm, m_run), jnp.where(better, ta, a_run)

    pm = jnp.full((RB, 1), jnp.inf, jnp.float32)
    pif = jnp.full((RB, 1), -1.0, jnp.float32)
    for k in range(K):
        m_run = jnp.full((RB, 1), -jnp.inf, jnp.float32)
        a_run = jnp.full((RB, 1), BIGF, jnp.float32)
        for j in range(SCT):
            s = sc_ref[:, j * CT:(j + 1) * CT]
            colgf = (lax.broadcasted_iota(jnp.int32, (1, CT), 1).astype(jnp.float32)
                     + _c0(j).astype(jnp.float32))
            m_run, a_run = peel_tile(s, jnp.broadcast_to(colgf, (RB, CT)),
                                     pm, pif, m_run, a_run)

        def tile_body(t, c):
            s, colgf = score_at(t * CT, True)
            return peel_tile(s, colgf, pm, pif, c[0], c[1])

        m, am = lax.fori_loop(t0 + SCT, t1, tile_body, (m_run, a_run))
        validb = m > -jnp.inf
        idx_ref[:, k:k + 1] = jnp.where(validb, am, rowgf).astype(jnp.int32)
        sm_ref[:, k:k + 1] = m
        pm, pif = m, am

    mall = sm_ref[:]
    validb = mall > -jnp.inf
    d = jnp.sqrt(jnp.maximum(-mall, 1e-12))
    d = jnp.where(validb, d, 1.0)
    cc_ref[:] = (0.5 * (jnp.cos(d * PI / CUT) + 1.0)) * validb.astype(jnp.float32)
    d_ref[:] = d


def _nbr(pos, batch, tlo, thi, interpret=False):
    posT = pos.T.reshape(3, N)
    bat2 = batch.reshape(N, 1)
    batT = batch.reshape(1, N)
    grid_spec = pltpu.PrefetchScalarGridSpec(
        num_scalar_prefetch=2,
        grid=(NB,),
        in_specs=[
            pl.BlockSpec((RB, 3), lambda b, *_: (b, 0)),
            pl.BlockSpec((3, N), lambda b, *_: (0, 0)),
            pl.BlockSpec((RB, 1), lambda b, *_: (b, 0)),
            pl.BlockSpec((1, N), lambda b, *_: (0, 0)),
        ],
        out_specs=[
            pl.BlockSpec((RB, K), lambda b, *_: (b, 0)),
            pl.BlockSpec((RB, K), lambda b, *_: (b, 0)),
            pl.BlockSpec((RB, K), lambda b, *_: (b, 0)),
        ],
        scratch_shapes=[pltpu.VMEM((RB, SCT * CT), jnp.float32),
                        pltpu.VMEM((RB, K), jnp.float32)],
    )
    return pl.pallas_call(
        _nbr_kernel,
        grid_spec=grid_spec,
        out_shape=[
            jax.ShapeDtypeStruct((N, K), jnp.int32),
            jax.ShapeDtypeStruct((N, K), jnp.float32),
            jax.ShapeDtypeStruct((N, K), jnp.float32),
        ],
        interpret=interpret,
    )(tlo, thi, pos, posT, bat2, batT)


_NBUF = 4


def _sc_gather(table, idx, D):
    B = idx.shape[0]
    bpw = B // _NW
    nch = bpw // _CH
    nsup = max(nch // _NBUF, 1)
    nb = min(_NBUF, nch)
    mesh = plsc.VectorSubcoreMesh(core_axis_name="c", subcore_axis_name="s")

    @functools.partial(
        pl.kernel,
        mesh=mesh,
        out_type=jax.ShapeDtypeStruct((B, D), jnp.float32),
        scratch_types=[
            pltpu.VMEM((bpw,), jnp.int32),
            pltpu.VMEM((nb, _CH, D), jnp.float32),
        ] + [pltpu.SemaphoreType.DMA] * nb,
    )
    def k(table_hbm, idx_hbm, out_hbm, idx_v, rows_v, *sems):
        wid = lax.axis_index("s") * _NC + lax.axis_index("c")
        base = wid * bpw
        pltpu.sync_copy(idx_hbm.at[pl.ds(base, bpw)], idx_v)

        def body(i, carry):
            cps = []
            for bidx in range(nb):
                ch = i * nb + bidx
                cps.append(pltpu.async_copy(
                    table_hbm.at[idx_v.at[pl.ds(ch * _CH, _CH)]],
                    rows_v.at[bidx], sems[bidx]))
            for bidx in range(nb):
                ch = i * nb + bidx
                cps[bidx].wait()
                pltpu.sync_copy(rows_v.at[bidx],
                                out_hbm.at[pl.ds(base + ch * _CH, _CH)])
            return carry

        lax.fori_loop(0, nsup, body, 0)

    return k(table, idx)


DP = 128

WROWS = 512
APW = N // _NW
ECH = 256
ACH = ECH // K
NCHK = APW // ACH


def _sc_agg(xl, wf, idx):
    wfl = wf.reshape(N * K, NF)
    mesh = plsc.VectorSubcoreMesh(core_axis_name="c", subcore_axis_name="s")

    @functools.partial(
        pl.kernel,
        mesh=mesh,
        out_type=jax.ShapeDtypeStruct((N, NF), jnp.float32),
        scratch_types=[
            pltpu.VMEM((WROWS, NF), jnp.float32),
            pltpu.VMEM((ECH, NF), jnp.float32),
            pltpu.VMEM((ECH,), jnp.int32),
            pltpu.VMEM((ACH, NF), jnp.float32),
        ],
    )
    def k(xl_hbm, wf_hbm, idx_hbm, agg_hbm, win_v, wch_v, idx_v, out_v):
        wid = lax.axis_index("s") * _NC + lax.axis_index("c")
        abase = wid * APW
        ebase = abase * K
        w0 = jnp.clip(abase - (WROWS - APW) // 2, 0, N - WROWS)
        w0 = pl.multiple_of(w0, 8)
        pltpu.sync_copy(xl_hbm.at[pl.ds(w0, WROWS)], win_v)

        def chunk(c, carry):
            e0 = pl.multiple_of(ebase + c * ECH, ECH)
            pltpu.sync_copy(idx_hbm.at[pl.ds(e0, ECH)], idx_v)
            pltpu.sync_copy(wf_hbm.at[pl.ds(e0, ECH)], wch_v)

            def atom(al, carry2):
                iv0 = idx_v[pl.ds(al * K, 16)]
                iv1 = idx_v[pl.ds(al * K + 16, 16)]
                accs = [jnp.zeros((16,), jnp.float32) for _ in range(NF // 16)]
                for kk in range(K):
                    col = iv0[kk] if kk < 16 else iv1[kk - 16]
                    off = col - w0
                    erow = al * K + kk
                    for f in range(NF // 16):
                        g = win_v[off, pl.ds(f * 16, 16)]
                        wv = wch_v[erow, pl.ds(f * 16, 16)]
                        accs[f] = accs[f] + g * wv
                for f in range(NF // 16):
                    out_v[al, pl.ds(f * 16, 16)] = accs[f]
                return carry2

            lax.fori_loop(0, ACH, atom, 0)
            a0 = pl.multiple_of(abase + c * ACH, ACH)
            pltpu.sync_copy(out_v, agg_hbm.at[pl.ds(a0, ACH)])
            return carry

        lax.fori_loop(0, NCHK, chunk, 0)

    return k(xl, wfl, idx)


def _red_kernel(w_ref, xg_ref, o_ref):
    o_ref[:] = jnp.sum(w_ref[:] * xg_ref[:][:, :, :NF], axis=1)


def _red(wf, xg, interpret=False):
    return pl.pallas_call(
        _red_kernel,
        grid=(N // BD,),
        in_specs=[
            pl.BlockSpec((BD, K, NF), lambda b: (b, 0, 0)),
            pl.BlockSpec((BD, K, DP), lambda b: (b, 0, 0)),
        ],
        out_specs=pl.BlockSpec((BD, NF), lambda b: (b, 0)),
        out_shape=jax.ShapeDtypeStruct((N, NF), jnp.float32),
        interpret=interpret,
    )(wf, xg)


def _wker_kernel(d_ref, cc_ref, w1_ref, b1_ref, w2_ref, b2_ref, w_ref):
    offs = lax.broadcasted_iota(jnp.int32, (1, NG), 1).astype(jnp.float32) * _STEP
    d = d_ref[:]
    ea = jnp.exp(_COEFF * (d - offs) ** 2)
    t = jnp.dot(ea, w1_ref[:], preferred_element_type=jnp.float32) + b1_ref[:]
    t = _ssp(t)
    w = jnp.dot(t, w2_ref[:], preferred_element_type=jnp.float32) + b2_ref[:]
    w_ref[:] = w.reshape(BD, K, NF) * cc_ref[:]


def _wker(d, cc3, w1, b1, w2, b2, interpret=False):
    full = lambda *shape: pl.BlockSpec(shape, lambda b: tuple(0 for _ in shape))
    return pl.pallas_call(
        _wker_kernel,
        grid=(N // BD,),
        in_specs=[
            pl.BlockSpec((BD * K, 1), lambda b: (b, 0)),
            pl.BlockSpec((BD, K, 1), lambda b: (b, 0, 0)),
            full(NG, NF), full(1, NF), full(NF, NF), full(1, NF),
        ],
        out_specs=pl.BlockSpec((BD, K, NF), lambda b: (b, 0, 0)),
        out_shape=jax.ShapeDtypeStruct((N, K, NF), jnp.float32),
        interpret=interpret,
    )(d, cc3, w1, b1.reshape(1, NF), w2, b2.reshape(1, NF))


def _upd2_kernel(agg_ref, h_ref, cf2w_ref, cf2b_ref, ilw_ref, ilb_ref, ho_ref):
    t2 = jnp.dot(agg_ref[:], cf2w_ref[:], preferred_element_type=jnp.float32) + cf2b_ref[:]
    t2 = _ssp(t2)
    xc = jnp.dot(t2, ilw_ref[:], preferred_element_type=jnp.float32) + ilb_ref[:]
    ho_ref[:] = h_ref[:][:, :H] + xc


def _upd2(agg, h, cf2w, cf2b, ilw, ilb, interpret=False):
    dh = h.shape[1]
    full = lambda *shape: pl.BlockSpec(shape, lambda b: tuple(0 for _ in shape))
    return pl.pallas_call(
        _upd2_kernel,
        grid=(N // BD,),
        in_specs=[
            pl.BlockSpec((BD, NF), lambda b: (b, 0)),
            pl.BlockSpec((BD, dh), lambda b: (b, 0)),
            full(NF, H), full(1, H), full(H, H), full(1, H),
        ],
        out_specs=pl.BlockSpec((BD, H), lambda b: (b, 0)),
        out_shape=jax.ShapeDtypeStruct((N, H), jnp.float32),
        interpret=interpret,
    )(agg, h, cf2w, cf2b.reshape(1, H), ilw, ilb.reshape(1, H))


def _lin_kernel(x_ref, w_ref, o_ref):
    o_ref[:] = jnp.dot(x_ref[:], w_ref[:], preferred_element_type=jnp.float32)


def _lin(x, w, interpret=False):
    n, dx = x.shape
    d2 = w.shape[1]
    return pl.pallas_call(
        _lin_kernel,
        out_shape=jax.ShapeDtypeStruct((n, d2), jnp.float32),
        interpret=interpret,
    )(x, w)


def _readout_kernel(h_ref, batT_ref, hw1_ref, hb1_ref, hw2_ref, hb2_ref,
                    ow_ref, ob_ref, o_ref):
    t = jnp.dot(h_ref[:], hw1_ref[:], preferred_element_type=jnp.float32) + hb1_ref[:]
    t = _ssp(t)
    t = jnp.dot(t, hw2_ref[:], preferred_element_type=jnp.float32) + hb2_ref[:]
    gids = lax.broadcasted_iota(jnp.int32, (G, 1), 0)
    maskf = (batT_ref[:] == gids).astype(jnp.float32)
    seg = jnp.dot(maskf, t, preferred_element_type=jnp.float32)
    o_ref[:] = seg * ow_ref[0, 0] + ob_ref[0, 0]


def _readout(h, batch, hw1p, hb1, hw2, hb2, ow, ob, interpret=False):
    return pl.pallas_call(
        _readout_kernel,
        out_shape=jax.ShapeDtypeStruct((G, 1), jnp.float32),
        interpret=interpret,
    )(h, batch.reshape(1, N), hw1p, hb1.reshape(1, H // 2),
      hw2, hb2.reshape(1, 1), ow, ob.reshape(1, 1))


def kernel(z, pos, batch, emb, mlp_w1, mlp_b1, mlp_w2, mlp_b2, cf1_w, cf2_w,
           cf2_b, il_w, il_b, hw1, hb1, hw2, hb2, ow, ob):
    batch = batch.astype(jnp.int32)
    z = z.astype(jnp.int32)

    starts = jnp.searchsorted(batch, jnp.arange(G + 1, dtype=jnp.int32),
                              side="left").astype(jnp.int32)
    r0 = jnp.arange(NB, dtype=jnp.int32) * RB
    g_lo = batch[r0]
    g_hi = batch[r0 + RB - 1]
    col_lo = starts[g_lo]
    col_hi = starts[g_hi + 1]
    tlo = col_lo // CT
    thi = (col_hi + CT - 1) // CT

    idx, cc, dmat = _nbr(pos, batch, tlo, thi)
    idxf = idx.reshape(N * K)
    cc3 = cc.reshape(N, K, 1)
    dflat = dmat.reshape(N * K, 1)

    rsub = jnp.arange(_NW, dtype=jnp.int32) * APW
    cmin = starts[batch[rsub]]
    cmax = starts[batch[rsub + APW - 1] + 1]
    w0s = jnp.clip(rsub - (WROWS - APW) // 2, 0, N - WROWS)
    fits = jnp.all((cmin >= w0s) & (cmax <= w0s + WROWS))

    emb_p = jnp.pad(emb, ((0, 0), (0, DP - H)))
    h = _sc_gather(emb_p, z, DP)

    def _agg_fast(ops):
        xl, wf, idxf = ops
        return _sc_agg(xl, wf, idxf)

    def _agg_slow(ops):
        xl, wf, idxf = ops
        xlp = jnp.pad(xl, ((0, 0), (0, DP - NF)))
        xg = _sc_gather(xlp, idxf, DP).reshape(N, K, DP)
        return _red(wf, xg)

    wfs = [_wker(dflat, cc3, mlp_w1[i], mlp_b1[i], mlp_w2[i], mlp_b2[i])
           for i in range(L)]
    for i in range(L):
        cf1p = jnp.pad(cf1_w[i], ((0, h.shape[1] - H), (0, 0)))
        xl = _lin(h, cf1p)
        agg = lax.cond(fits, _agg_fast, _agg_slow, (xl, wfs[i], idxf))
        h = _upd2(agg, h, cf2_w[i], cf2_b[i], il_w[i], il_b[i])
    return _readout(h, batch, hw1, hb1, hw2, hb2, ow, ob)


# speedup vs baseline: 11.2697x; 1.1864x over previous
import functools

import jax
import jax.numpy as jnp
import numpy as np
from jax import lax
from jax.experimental import pallas as pl
from jax.experimental.pallas import tpu as pltpu
from jax.experimental.pallas import tpu_sc as plsc

N = 4096
G = 128
K = 32
H = 64
NF = 64
NG = 50
CUT = 10.0
L = 3
LOG2 = float(np.log(2.0))

RB = 128
CT = 128
NB = N // RB
BD = 512
PI = float(np.pi)

_OFFS = np.linspace(0.0, CUT, NG).astype(np.float32)
_STEP = np.float32(_OFFS[1] - _OFFS[0])
_COEFF = np.float32(-0.5 / (_STEP * _STEP))

_NC = 2
_NS = 16
_NW = _NC * _NS
_CH = 128


def _ssp(x):
    return jax.nn.softplus(x) - LOG2


SCT = 3


def _nbr_kernel(tlo_ref, thi_ref, pos_ref, posT_ref, bat_ref, batT_ref,
                idx_ref, cc_ref, d_ref, sc_ref, sm_ref):
    b = pl.program_id(0)
    t0 = tlo_ref[b]
    t1 = thi_ref[b]

    xr = pos_ref[:, 0:1]
    yr = pos_ref[:, 1:2]
    zr = pos_ref[:, 2:3]
    sqr = (xr * xr + yr * yr) + zr * zr
    br = bat_ref[:]
    rowg = lax.broadcasted_iota(jnp.int32, (RB, 1), 0) + b * RB
    rowgf = rowg.astype(jnp.float32)
    BIGF = jnp.float32(N)

    def score_at(c0, tvalid):
        xc = posT_ref[0:1, pl.ds(c0, CT)]
        yc = posT_ref[1:2, pl.ds(c0, CT)]
        zc = posT_ref[2:3, pl.ds(c0, CT)]
        sqc = (xc * xc + yc * yc) + zc * zc
        bc = batT_ref[0:1, pl.ds(c0, CT)]
        colg = lax.broadcasted_iota(jnp.int32, (1, CT), 1) + c0
        dot = (xr * xc + yr * yc) + zr * zc
        d2 = sqr + sqc - 2.0 * dot
        m = (br == bc) & (rowg != colg) & (d2 <= CUT * CUT) & tvalid
        s = jnp.where(m, -d2, -jnp.inf)
        colgf = (lax.broadcasted_iota(jnp.int32, (1, CT), 1).astype(jnp.float32)
                 + c0.astype(jnp.float32))
        return s, jnp.broadcast_to(colgf, (RB, CT))

    def _c0(j):
        return jnp.minimum((t0 + j) * CT, N - CT)

    for j in range(SCT):
        s, _ = score_at(_c0(j), (t0 + j) < t1)
        sc_ref[:, j * CT:(j + 1) * CT] = s

    def peel_tile(s, colgf, pm, pif, m_run, a_run):
        elig = (s < pm) | ((s == pm) & (colgf > pif))
        s2 = jnp.where(elig, s, -jnp.inf)
        tm = jnp.max(s2, axis=1, keepdims=True)
        ta = jnp.min(jnp.where(s2 == tm, colgf, BIGF), axis=1, keepdims=True)
        better = (tm > m_run) | ((tm == m_run) & (ta < a_run))
        return jnp.where(better, tm, m_run), jnp.where(better, ta, a_run)

    pm = jnp.full((RB, 1), jnp.inf, jnp.float32)
    pif = jnp.full((RB, 1), -1.0, jnp.float32)
    for k in range(K):
        m_run = jnp.full((RB, 1), -jnp.inf, jnp.float32)
        a_run = jnp.full((RB, 1), BIGF, jnp.float32)
        for j in range(SCT):
            s = sc_ref[:, j * CT:(j + 1) * CT]
            colgf = (lax.broadcasted_iota(jnp.int32, (1, CT), 1).astype(jnp.float32)
                     + _c0(j).astype(jnp.float32))
            m_run, a_run = peel_tile(s, jnp.broadcast_to(colgf, (RB, CT)),
                                     pm, pif, m_run, a_run)

        def tile_body(t, c):
            s, colgf = score_at(t * CT, True)
            return peel_tile(s, colgf, pm, pif, c[0], c[1])

        m, am = lax.fori_loop(t0 + SCT, t1, tile_body, (m_run, a_run))
        validb = m > -jnp.inf
        idx_ref[:, k:k + 1] = jnp.where(validb, am, rowgf).astype(jnp.int32)
        sm_ref[:, k:k + 1] = m
        pm, pif = m, am

    mall = sm_ref[:]
    validb = mall > -jnp.inf
    d = jnp.sqrt(jnp.maximum(-mall, 1e-12))
    d = jnp.where(validb, d, 1.0)
    cc_ref[:] = (0.5 * (jnp.cos(d * PI / CUT) + 1.0)) * validb.astype(jnp.float32)
    d_ref[:] = d


def _nbr(pos, batch, tlo, thi, interpret=False):
    posT = pos.T.reshape(3, N)
    bat2 = batch.reshape(N, 1)
    batT = batch.reshape(1, N)
    grid_spec = pltpu.PrefetchScalarGridSpec(
        num_scalar_prefetch=2,
        grid=(NB,),
        in_specs=[
            pl.BlockSpec((RB, 3), lambda b, *_: (b, 0)),
            pl.BlockSpec((3, N), lambda b, *_: (0, 0)),
            pl.BlockSpec((RB, 1), lambda b, *_: (b, 0)),
            pl.BlockSpec((1, N), lambda b, *_: (0, 0)),
        ],
        out_specs=[
            pl.BlockSpec((RB, K), lambda b, *_: (b, 0)),
            pl.BlockSpec((RB, K), lambda b, *_: (b, 0)),
            pl.BlockSpec((RB, K), lambda b, *_: (b, 0)),
        ],
        scratch_shapes=[pltpu.VMEM((RB, SCT * CT), jnp.float32),
                        pltpu.VMEM((RB, K), jnp.float32)],
    )
    return pl.pallas_call(
        _nbr_kernel,
        grid_spec=grid_spec,
        out_shape=[
            jax.ShapeDtypeStruct((N, K), jnp.int32),
            jax.ShapeDtypeStruct((N, K), jnp.float32),
            jax.ShapeDtypeStruct((N, K), jnp.float32),
        ],
        interpret=interpret,
    )(tlo, thi, pos, posT, bat2, batT)


_NBUF = 4


def _sc_gather(table, idx, D):
    B = idx.shape[0]
    bpw = B // _NW
    nch = bpw // _CH
    nsup = max(nch // _NBUF, 1)
    nb = min(_NBUF, nch)
    mesh = plsc.VectorSubcoreMesh(core_axis_name="c", subcore_axis_name="s")

    @functools.partial(
        pl.kernel,
        mesh=mesh,
        out_type=jax.ShapeDtypeStruct((B, D), jnp.float32),
        scratch_types=[
            pltpu.VMEM((bpw,), jnp.int32),
            pltpu.VMEM((nb, _CH, D), jnp.float32),
        ] + [pltpu.SemaphoreType.DMA] * nb,
    )
    def k(table_hbm, idx_hbm, out_hbm, idx_v, rows_v, *sems):
        wid = lax.axis_index("s") * _NC + lax.axis_index("c")
        base = wid * bpw
        pltpu.sync_copy(idx_hbm.at[pl.ds(base, bpw)], idx_v)

        def body(i, carry):
            cps = []
            for bidx in range(nb):
                ch = i * nb + bidx
                cps.append(pltpu.async_copy(
                    table_hbm.at[idx_v.at[pl.ds(ch * _CH, _CH)]],
                    rows_v.at[bidx], sems[bidx]))
            for bidx in range(nb):
                ch = i * nb + bidx
                cps[bidx].wait()
                pltpu.sync_copy(rows_v.at[bidx],
                                out_hbm.at[pl.ds(base + ch * _CH, _CH)])
            return carry

        lax.fori_loop(0, nsup, body, 0)

    return k(table, idx)


DP = 128

WROWS = 448
APW = N // _NW
ECH = 256
ACH = ECH // K
NCHK = APW // ACH


def _sc_agg(xl, wf, idx):
    wfl = wf.reshape(N * K, NF)
    mesh = plsc.VectorSubcoreMesh(core_axis_name="c", subcore_axis_name="s")

    @functools.partial(
        pl.kernel,
        mesh=mesh,
        out_type=jax.ShapeDtypeStruct((N, NF), jnp.float32),
        scratch_types=[
            pltpu.VMEM((WROWS, NF), jnp.float32),
            pltpu.VMEM((2, ECH, NF), jnp.float32),
            pltpu.VMEM((2, ECH), jnp.int32),
            pltpu.VMEM((ACH, NF), jnp.float32),
            pltpu.SemaphoreType.DMA,
            pltpu.SemaphoreType.DMA,
            pltpu.SemaphoreType.DMA,
            pltpu.SemaphoreType.DMA,
        ],
    )
    def k(xl_hbm, wf_hbm, idx_hbm, agg_hbm, win_v, wch_v, idx_v, out_v,
          si0, si1, sw0, sw1):
        wid = lax.axis_index("s") * _NC + lax.axis_index("c")
        abase = wid * APW
        ebase = abase * K
        siv = (si0, si1)
        swv = (sw0, sw1)
        w0 = jnp.clip(abase - (WROWS - APW) // 2, 0, N - WROWS)
        w0 = pl.multiple_of(w0, 8)
        pltpu.sync_copy(xl_hbm.at[pl.ds(w0, WROWS)], win_v)

        def issue(c, slot):
            e0 = pl.multiple_of(
                jnp.minimum(ebase + c * ECH, N * K - ECH), ECH)
            pltpu.async_copy(idx_hbm.at[pl.ds(e0, ECH)], idx_v.at[slot],
                             siv[slot])
            pltpu.async_copy(wf_hbm.at[pl.ds(e0, ECH)], wch_v.at[slot],
                             swv[slot])

        def drain(slot):
            pltpu.make_async_copy(idx_hbm.at[pl.ds(0, ECH)], idx_v.at[slot],
                                  siv[slot]).wait()
            pltpu.make_async_copy(wf_hbm.at[pl.ds(0, ECH)], wch_v.at[slot],
                                  swv[slot]).wait()

        issue(0, 0)
        issue(1, 1)

        def process(c, slot):
            drain(slot)

            def atom(al, carry2):
                iv0 = idx_v[slot, pl.ds(al * K, 16)]
                iv1 = idx_v[slot, pl.ds(al * K + 16, 16)]
                accs = [jnp.zeros((16,), jnp.float32) for _ in range(NF // 16)]
                for kk in range(K):
                    col = iv0[kk] if kk < 16 else iv1[kk - 16]
                    off = col - w0
                    erow = al * K + kk
                    for f in range(NF // 16):
                        g = win_v[off, pl.ds(f * 16, 16)]
                        wv = wch_v[slot, erow, pl.ds(f * 16, 16)]
                        accs[f] = accs[f] + g * wv
                for f in range(NF // 16):
                    out_v[al, pl.ds(f * 16, 16)] = accs[f]
                return carry2

            lax.fori_loop(0, ACH, atom, 0)
            a0 = pl.multiple_of(abase + c * ACH, ACH)
            pltpu.sync_copy(out_v, agg_hbm.at[pl.ds(a0, ACH)])
            issue(c + 2, slot)

        def sup(s_, carry):
            process(s_ * 2, 0)
            process(s_ * 2 + 1, 1)
            return carry

        lax.fori_loop(0, NCHK // 2, sup, 0)
        drain(0)
        drain(1)

    return k(xl, wfl, idx)


def _red_kernel(w_ref, xg_ref, o_ref):
    o_ref[:] = jnp.sum(w_ref[:] * xg_ref[:][:, :, :NF], axis=1)


def _red(wf, xg, interpret=False):
    return pl.pallas_call(
        _red_kernel,
        grid=(N // BD,),
        in_specs=[
            pl.BlockSpec((BD, K, NF), lambda b: (b, 0, 0)),
            pl.BlockSpec((BD, K, DP), lambda b: (b, 0, 0)),
        ],
        out_specs=pl.BlockSpec((BD, NF), lambda b: (b, 0)),
        out_shape=jax.ShapeDtypeStruct((N, NF), jnp.float32),
        interpret=interpret,
    )(wf, xg)


def _wker_kernel(d_ref, cc_ref, w1_ref, b1_ref, w2_ref, b2_ref, w_ref):
    offs = lax.broadcasted_iota(jnp.int32, (1, NG), 1).astype(jnp.float32) * _STEP
    d = d_ref[:]
    ea = jnp.exp(_COEFF * (d - offs) ** 2)
    t = jnp.dot(ea, w1_ref[:], preferred_element_type=jnp.float32) + b1_ref[:]
    t = _ssp(t)
    w = jnp.dot(t, w2_ref[:], preferred_element_type=jnp.float32) + b2_ref[:]
    w_ref[:] = w.reshape(BD, K, NF) * cc_ref[:]


def _wker(d, cc3, w1, b1, w2, b2, interpret=False):
    full = lambda *shape: pl.BlockSpec(shape, lambda b: tuple(0 for _ in shape))
    return pl.pallas_call(
        _wker_kernel,
        grid=(N // BD,),
        in_specs=[
            pl.BlockSpec((BD * K, 1), lambda b: (b, 0)),
            pl.BlockSpec((BD, K, 1), lambda b: (b, 0, 0)),
            full(NG, NF), full(1, NF), full(NF, NF), full(1, NF),
        ],
        out_specs=pl.BlockSpec((BD, K, NF), lambda b: (b, 0, 0)),
        out_shape=jax.ShapeDtypeStruct((N, K, NF), jnp.float32),
        interpret=interpret,
    )(d, cc3, w1, b1.reshape(1, NF), w2, b2.reshape(1, NF))


def _upd2_kernel(agg_ref, h_ref, cf2w_ref, cf2b_ref, ilw_ref, ilb_ref, ho_ref):
    t2 = jnp.dot(agg_ref[:], cf2w_ref[:], preferred_element_type=jnp.float32) + cf2b_ref[:]
    t2 = _ssp(t2)
    xc = jnp.dot(t2, ilw_ref[:], preferred_element_type=jnp.float32) + ilb_ref[:]
    ho_ref[:] = h_ref[:][:, :H] + xc


def _upd2(agg, h, cf2w, cf2b, ilw, ilb, interpret=False):
    dh = h.shape[1]
    full = lambda *shape: pl.BlockSpec(shape, lambda b: tuple(0 for _ in shape))
    return pl.pallas_call(
        _upd2_kernel,
        grid=(N // BD,),
        in_specs=[
            pl.BlockSpec((BD, NF), lambda b: (b, 0)),
            pl.BlockSpec((BD, dh), lambda b: (b, 0)),
            full(NF, H), full(1, H), full(H, H), full(1, H),
        ],
        out_specs=pl.BlockSpec((BD, H), lambda b: (b, 0)),
        out_shape=jax.ShapeDtypeStruct((N, H), jnp.float32),
        interpret=interpret,
    )(agg, h, cf2w, cf2b.reshape(1, H), ilw, ilb.reshape(1, H))


def _lin_kernel(x_ref, w_ref, o_ref):
    o_ref[:] = jnp.dot(x_ref[:], w_ref[:], preferred_element_type=jnp.float32)


def _lin(x, w, interpret=False):
    n, dx = x.shape
    d2 = w.shape[1]
    return pl.pallas_call(
        _lin_kernel,
        out_shape=jax.ShapeDtypeStruct((n, d2), jnp.float32),
        interpret=interpret,
    )(x, w)


def _readout_kernel(h_ref, batT_ref, hw1_ref, hb1_ref, hw2_ref, hb2_ref,
                    ow_ref, ob_ref, o_ref):
    t = jnp.dot(h_ref[:], hw1_ref[:], preferred_element_type=jnp.float32) + hb1_ref[:]
    t = _ssp(t)
    t = jnp.dot(t, hw2_ref[:], preferred_element_type=jnp.float32) + hb2_ref[:]
    gids = lax.broadcasted_iota(jnp.int32, (G, 1), 0)
    maskf = (batT_ref[:] == gids).astype(jnp.float32)
    seg = jnp.dot(maskf, t, preferred_element_type=jnp.float32)
    o_ref[:] = seg * ow_ref[0, 0] + ob_ref[0, 0]


def _readout(h, batch, hw1p, hb1, hw2, hb2, ow, ob, interpret=False):
    return pl.pallas_call(
        _readout_kernel,
        out_shape=jax.ShapeDtypeStruct((G, 1), jnp.float32),
        interpret=interpret,
    )(h, batch.reshape(1, N), hw1p, hb1.reshape(1, H // 2),
      hw2, hb2.reshape(1, 1), ow, ob.reshape(1, 1))


def kernel(z, pos, batch, emb, mlp_w1, mlp_b1, mlp_w2, mlp_b2, cf1_w, cf2_w,
           cf2_b, il_w, il_b, hw1, hb1, hw2, hb2, ow, ob):
    batch = batch.astype(jnp.int32)
    z = z.astype(jnp.int32)

    starts = jnp.searchsorted(batch, jnp.arange(G + 1, dtype=jnp.int32),
                              side="left").astype(jnp.int32)
    r0 = jnp.arange(NB, dtype=jnp.int32) * RB
    g_lo = batch[r0]
    g_hi = batch[r0 + RB - 1]
    col_lo = starts[g_lo]
    col_hi = starts[g_hi + 1]
    tlo = col_lo // CT
    thi = (col_hi + CT - 1) // CT

    idx, cc, dmat = _nbr(pos, batch, tlo, thi)
    idxf = idx.reshape(N * K)
    cc3 = cc.reshape(N, K, 1)
    dflat = dmat.reshape(N * K, 1)

    rsub = jnp.arange(_NW, dtype=jnp.int32) * APW
    cmin = starts[batch[rsub]]
    cmax = starts[batch[rsub + APW - 1] + 1]
    w0s = jnp.clip(rsub - (WROWS - APW) // 2, 0, N - WROWS)
    fits = jnp.all((cmin >= w0s) & (cmax <= w0s + WROWS))

    emb_p = jnp.pad(emb, ((0, 0), (0, DP - H)))
    h = _sc_gather(emb_p, z, DP)

    def _agg_fast(ops):
        xl, wf, idxf = ops
        return _sc_agg(xl, wf, idxf)

    def _agg_slow(ops):
        xl, wf, idxf = ops
        xlp = jnp.pad(xl, ((0, 0), (0, DP - NF)))
        xg = _sc_gather(xlp, idxf, DP).reshape(N, K, DP)
        return _red(wf, xg)

    wfs = [_wker(dflat, cc3, mlp_w1[i], mlp_b1[i], mlp_w2[i], mlp_b2[i])
           for i in range(L)]
    for i in range(L):
        cf1p = jnp.pad(cf1_w[i], ((0, h.shape[1] - H), (0, 0)))
        xl = _lin(h, cf1p)
        agg = lax.cond(fits, _agg_fast, _agg_slow, (xl, wfs[i], idxf))
        h = _upd2(agg, h, cf2_w[i], cf2_b[i], il_w[i], il_b[i])
    return _readout(h, batch, hw1, hb1, hw2, hb2, ow, ob)


# speedup vs baseline: 11.2742x; 1.0004x over previous
import functools

import jax
import jax.numpy as jnp
import numpy as np
from jax import lax
from jax.experimental import pallas as pl
from jax.experimental.pallas import tpu as pltpu
from jax.experimental.pallas import tpu_sc as plsc

N = 4096
G = 128
K = 32
H = 64
NF = 64
NG = 50
CUT = 10.0
L = 3
LOG2 = float(np.log(2.0))

RB = 128
CT = 128
NB = N // RB
BD = 512
PI = float(np.pi)

_OFFS = np.linspace(0.0, CUT, NG).astype(np.float32)
_STEP = np.float32(_OFFS[1] - _OFFS[0])
_COEFF = np.float32(-0.5 / (_STEP * _STEP))

_NC = 2
_NS = 16
_NW = _NC * _NS
_CH = 128


def _ssp(x):
    return jax.nn.softplus(x) - LOG2


SCT = 3


def _nbr_kernel(tlo_ref, thi_ref, pos_ref, posT_ref, bat_ref, batT_ref,
                idx_ref, cc_ref, d_ref, sc_ref, sm_ref):
    b = pl.program_id(0)
    t0 = tlo_ref[b]
    t1 = thi_ref[b]

    xr = pos_ref[:, 0:1]
    yr = pos_ref[:, 1:2]
    zr = pos_ref[:, 2:3]
    sqr = (xr * xr + yr * yr) + zr * zr
    br = bat_ref[:]
    rowg = lax.broadcasted_iota(jnp.int32, (RB, 1), 0) + b * RB
    rowgf = rowg.astype(jnp.float32)
    BIGF = jnp.float32(N)

    def score_at(c0, tvalid):
        xc = posT_ref[0:1, pl.ds(c0, CT)]
        yc = posT_ref[1:2, pl.ds(c0, CT)]
        zc = posT_ref[2:3, pl.ds(c0, CT)]
        sqc = (xc * xc + yc * yc) + zc * zc
        bc = batT_ref[0:1, pl.ds(c0, CT)]
        colg = lax.broadcasted_iota(jnp.int32, (1, CT), 1) + c0
        dot = (xr * xc + yr * yc) + zr * zc
        d2 = sqr + sqc - 2.0 * dot
        m = (br == bc) & (rowg != colg) & (d2 <= CUT * CUT) & tvalid
        s = jnp.where(m, -d2, -jnp.inf)
        colgf = (lax.broadcasted_iota(jnp.int32, (1, CT), 1).astype(jnp.float32)
                 + c0.astype(jnp.float32))
        return s, jnp.broadcast_to(colgf, (RB, CT))

    def _c0(j):
        return jnp.minimum((t0 + j) * CT, N - CT)

    for j in range(SCT):
        s, _ = score_at(_c0(j), (t0 + j) < t1)
        sc_ref[:, j * CT:(j + 1) * CT] = s

    def peel_tile(s, colgf, pm, pif, m_run, a_run):
        elig = (s < pm) | ((s == pm) & (colgf > pif))
        s2 = jnp.where(elig, s, -jnp.inf)
        tm = jnp.max(s2, axis=1, keepdims=True)
        ta = jnp.min(jnp.where(s2 == tm, colgf, BIGF), axis=1, keepdims=True)
        better = (tm > m_run) | ((tm == m_run) & (ta < a_run))
        return jnp.where(better, tm, m_run), jnp.where(better, ta, a_run)

    pm = jnp.full((RB, 1), jnp.inf, jnp.float32)
    pif = jnp.full((RB, 1), -1.0, jnp.float32)
    for k in range(K):
        m_run = jnp.full((RB, 1), -jnp.inf, jnp.float32)
        a_run = jnp.full((RB, 1), BIGF, jnp.float32)
        for j in range(SCT):
            s = sc_ref[:, j * CT:(j + 1) * CT]
            colgf = (lax.broadcasted_iota(jnp.int32, (1, CT), 1).astype(jnp.float32)
                     + _c0(j).astype(jnp.float32))
            m_run, a_run = peel_tile(s, jnp.broadcast_to(colgf, (RB, CT)),
                                     pm, pif, m_run, a_run)

        def tile_body(t, c):
            s, colgf = score_at(t * CT, True)
            return peel_tile(s, colgf, pm, pif, c[0], c[1])

        m, am = lax.fori_loop(t0 + SCT, t1, tile_body, (m_run, a_run))
        validb = m > -jnp.inf
        idx_ref[:, k:k + 1] = jnp.where(validb, am, rowgf).astype(jnp.int32)
        sm_ref[:, k:k + 1] = m
        pm, pif = m, am

    mall = sm_ref[:]
    validb = mall > -jnp.inf
    d = jnp.sqrt(jnp.maximum(-mall, 1e-12))
    d = jnp.where(validb, d, 1.0)
    cc_ref[:] = (0.5 * (jnp.cos(d * PI / CUT) + 1.0)) * validb.astype(jnp.float32)
    d_ref[:] = d


def _nbr(pos, batch, tlo, thi):
    posT = pos.T.reshape(3, N)
    bat2 = batch.reshape(N, 1)
    batT = batch.reshape(1, N)
    grid_spec = pltpu.PrefetchScalarGridSpec(
        num_scalar_prefetch=2,
        grid=(NB,),
        in_specs=[
            pl.BlockSpec((RB, 3), lambda b, *_: (b, 0)),
            pl.BlockSpec((3, N), lambda b, *_: (0, 0)),
            pl.BlockSpec((RB, 1), lambda b, *_: (b, 0)),
            pl.BlockSpec((1, N), lambda b, *_: (0, 0)),
        ],
        out_specs=[
            pl.BlockSpec((RB, K), lambda b, *_: (b, 0)),
            pl.BlockSpec((RB, K), lambda b, *_: (b, 0)),
            pl.BlockSpec((RB, K), lambda b, *_: (b, 0)),
        ],
        scratch_shapes=[pltpu.VMEM((RB, SCT * CT), jnp.float32),
                        pltpu.VMEM((RB, K), jnp.float32)],
    )
    return pl.pallas_call(
        _nbr_kernel,
        grid_spec=grid_spec,
        out_shape=[
            jax.ShapeDtypeStruct((N, K), jnp.int32),
            jax.ShapeDtypeStruct((N, K), jnp.float32),
            jax.ShapeDtypeStruct((N, K), jnp.float32),
        ],
    )(tlo, thi, pos, posT, bat2, batT)


_NBUF = 4


def _sc_gather(table, idx, D):
    B = idx.shape[0]
    bpw = B // _NW
    nch = bpw // _CH
    nsup = max(nch // _NBUF, 1)
    nb = min(_NBUF, nch)
    mesh = plsc.VectorSubcoreMesh(core_axis_name="c", subcore_axis_name="s")

    @functools.partial(
        pl.kernel,
        mesh=mesh,
        out_type=jax.ShapeDtypeStruct((B, D), jnp.float32),
        scratch_types=[
            pltpu.VMEM((bpw,), jnp.int32),
            pltpu.VMEM((nb, _CH, D), jnp.float32),
        ] + [pltpu.SemaphoreType.DMA] * nb,
    )
    def k(table_hbm, idx_hbm, out_hbm, idx_v, rows_v, *sems):
        wid = lax.axis_index("s") * _NC + lax.axis_index("c")
        base = wid * bpw
        pltpu.sync_copy(idx_hbm.at[pl.ds(base, bpw)], idx_v)

        def body(i, carry):
            cps = []
            for bidx in range(nb):
                ch = i * nb + bidx
                cps.append(pltpu.async_copy(
                    table_hbm.at[idx_v.at[pl.ds(ch * _CH, _CH)]],
                    rows_v.at[bidx], sems[bidx]))
            for bidx in range(nb):
                ch = i * nb + bidx
                cps[bidx].wait()
                pltpu.sync_copy(rows_v.at[bidx],
                                out_hbm.at[pl.ds(base + ch * _CH, _CH)])
            return carry

        lax.fori_loop(0, nsup, body, 0)

    return k(table, idx)


DP = 128

WROWS = 448
APW = N // _NW
ECH = 256
ACH = ECH // K
NCHK = APW // ACH


def _sc_agg(xl, wf, idx):
    wfl = wf.reshape(N * K, NF)
    mesh = plsc.VectorSubcoreMesh(core_axis_name="c", subcore_axis_name="s")

    @functools.partial(
        pl.kernel,
        mesh=mesh,
        out_type=jax.ShapeDtypeStruct((N, NF), jnp.float32),
        scratch_types=[
            pltpu.VMEM((WROWS, NF), jnp.float32),
            pltpu.VMEM((2, ECH, NF), jnp.float32),
            pltpu.VMEM((2, ECH), jnp.int32),
            pltpu.VMEM((ACH, NF), jnp.float32),
            pltpu.SemaphoreType.DMA,
            pltpu.SemaphoreType.DMA,
            pltpu.SemaphoreType.DMA,
            pltpu.SemaphoreType.DMA,
        ],
    )
    def k(xl_hbm, wf_hbm, idx_hbm, agg_hbm, win_v, wch_v, idx_v, out_v,
          si0, si1, sw0, sw1):
        wid = lax.axis_index("s") * _NC + lax.axis_index("c")
        abase = wid * APW
        ebase = abase * K
        siv = (si0, si1)
        swv = (sw0, sw1)
        w0 = jnp.clip(abase - (WROWS - APW) // 2, 0, N - WROWS)
        w0 = pl.multiple_of(w0, 8)
        pltpu.sync_copy(xl_hbm.at[pl.ds(w0, WROWS)], win_v)

        def issue(c, slot):
            e0 = pl.multiple_of(
                jnp.minimum(ebase + c * ECH, N * K - ECH), ECH)
            pltpu.async_copy(idx_hbm.at[pl.ds(e0, ECH)], idx_v.at[slot],
                             siv[slot])
            pltpu.async_copy(wf_hbm.at[pl.ds(e0, ECH)], wch_v.at[slot],
                             swv[slot])

        def drain(slot):
            pltpu.make_async_copy(idx_hbm.at[pl.ds(0, ECH)], idx_v.at[slot],
                                  siv[slot]).wait()
            pltpu.make_async_copy(wf_hbm.at[pl.ds(0, ECH)], wch_v.at[slot],
                                  swv[slot]).wait()

        issue(0, 0)
        issue(1, 1)

        def process(c, slot):
            drain(slot)

            def atom(al, carry2):
                iv0 = idx_v[slot, pl.ds(al * K, 16)]
                iv1 = idx_v[slot, pl.ds(al * K + 16, 16)]
                accs = [jnp.zeros((16,), jnp.float32) for _ in range(NF // 16)]
                for kk in range(K):
                    col = iv0[kk] if kk < 16 else iv1[kk - 16]
                    off = col - w0
                    erow = al * K + kk
                    for f in range(NF // 16):
                        g = win_v[off, pl.ds(f * 16, 16)]
                        wv = wch_v[slot, erow, pl.ds(f * 16, 16)]
                        accs[f] = accs[f] + g * wv
                for f in range(NF // 16):
                    out_v[al, pl.ds(f * 16, 16)] = accs[f]
                return carry2

            lax.fori_loop(0, ACH, atom, 0)
            a0 = pl.multiple_of(abase + c * ACH, ACH)
            pltpu.sync_copy(out_v, agg_hbm.at[pl.ds(a0, ACH)])
            issue(c + 2, slot)

        def sup(s_, carry):
            process(s_ * 2, 0)
            process(s_ * 2 + 1, 1)
            return carry

        lax.fori_loop(0, NCHK // 2, sup, 0)
        drain(0)
        drain(1)

    return k(xl, wfl, idx)


def _red_kernel(w_ref, xg_ref, o_ref):
    o_ref[:] = jnp.sum(w_ref[:] * xg_ref[:][:, :, :NF], axis=1)


def _red(wf, xg):
    return pl.pallas_call(
        _red_kernel,
        grid=(N // BD,),
        in_specs=[
            pl.BlockSpec((BD, K, NF), lambda b: (b, 0, 0)),
            pl.BlockSpec((BD, K, DP), lambda b: (b, 0, 0)),
        ],
        out_specs=pl.BlockSpec((BD, NF), lambda b: (b, 0)),
        out_shape=jax.ShapeDtypeStruct((N, NF), jnp.float32),
    )(wf, xg)


def _wker_kernel(d_ref, cc_ref, w1_ref, b1_ref, w2_ref, b2_ref, w_ref):
    offs = lax.broadcasted_iota(jnp.int32, (1, NG), 1).astype(jnp.float32) * _STEP
    d = d_ref[:]
    ea = jnp.exp(_COEFF * (d - offs) ** 2)
    t = jnp.dot(ea, w1_ref[:], preferred_element_type=jnp.float32) + b1_ref[:]
    t = _ssp(t)
    w = jnp.dot(t, w2_ref[:], preferred_element_type=jnp.float32) + b2_ref[:]
    w_ref[:] = w.reshape(BD, K, NF) * cc_ref[:]


def _wker(d, cc3, w1, b1, w2, b2):
    full = lambda *shape: pl.BlockSpec(shape, lambda b: tuple(0 for _ in shape))
    return pl.pallas_call(
        _wker_kernel,
        grid=(N // BD,),
        in_specs=[
            pl.BlockSpec((BD * K, 1), lambda b: (b, 0)),
            pl.BlockSpec((BD, K, 1), lambda b: (b, 0, 0)),
            full(NG, NF), full(1, NF), full(NF, NF), full(1, NF),
        ],
        out_specs=pl.BlockSpec((BD, K, NF), lambda b: (b, 0, 0)),
        out_shape=jax.ShapeDtypeStruct((N, K, NF), jnp.float32),
    )(d, cc3, w1, b1.reshape(1, NF), w2, b2.reshape(1, NF))


def _upd2_kernel(agg_ref, h_ref, cf2w_ref, cf2b_ref, ilw_ref, ilb_ref, ho_ref):
    t2 = jnp.dot(agg_ref[:], cf2w_ref[:], preferred_element_type=jnp.float32) + cf2b_ref[:]
    t2 = _ssp(t2)
    xc = jnp.dot(t2, ilw_ref[:], preferred_element_type=jnp.float32) + ilb_ref[:]
    ho_ref[:] = h_ref[:][:, :H] + xc


def _upd2(agg, h, cf2w, cf2b, ilw, ilb):
    dh = h.shape[1]
    full = lambda *shape: pl.BlockSpec(shape, lambda b: tuple(0 for _ in shape))
    return pl.pallas_call(
        _upd2_kernel,
        grid=(N // BD,),
        in_specs=[
            pl.BlockSpec((BD, NF), lambda b: (b, 0)),
            pl.BlockSpec((BD, dh), lambda b: (b, 0)),
            full(NF, H), full(1, H), full(H, H), full(1, H),
        ],
        out_specs=pl.BlockSpec((BD, H), lambda b: (b, 0)),
        out_shape=jax.ShapeDtypeStruct((N, H), jnp.float32),
    )(agg, h, cf2w, cf2b.reshape(1, H), ilw, ilb.reshape(1, H))


def _lin_kernel(x_ref, w_ref, o_ref):
    o_ref[:] = jnp.dot(x_ref[:], w_ref[:], preferred_element_type=jnp.float32)


def _lin(x, w):
    n, dx = x.shape
    d2 = w.shape[1]
    return pl.pallas_call(
        _lin_kernel,
        out_shape=jax.ShapeDtypeStruct((n, d2), jnp.float32),
    )(x, w)


def _readout_kernel(h_ref, batT_ref, hw1_ref, hb1_ref, hw2_ref, hb2_ref,
                    ow_ref, ob_ref, o_ref):
    t = jnp.dot(h_ref[:], hw1_ref[:], preferred_element_type=jnp.float32) + hb1_ref[:]
    t = _ssp(t)
    t = jnp.dot(t, hw2_ref[:], preferred_element_type=jnp.float32) + hb2_ref[:]
    gids = lax.broadcasted_iota(jnp.int32, (G, 1), 0)
    maskf = (batT_ref[:] == gids).astype(jnp.float32)
    seg = jnp.dot(maskf, t, preferred_element_type=jnp.float32)
    o_ref[:] = seg * ow_ref[0, 0] + ob_ref[0, 0]


def _readout(h, batch, hw1p, hb1, hw2, hb2, ow, ob):
    return pl.pallas_call(
        _readout_kernel,
        out_shape=jax.ShapeDtypeStruct((G, 1), jnp.float32),
    )(h, batch.reshape(1, N), hw1p, hb1.reshape(1, H // 2),
      hw2, hb2.reshape(1, 1), ow, ob.reshape(1, 1))


def kernel(z, pos, batch, emb, mlp_w1, mlp_b1, mlp_w2, mlp_b2, cf1_w, cf2_w,
           cf2_b, il_w, il_b, hw1, hb1, hw2, hb2, ow, ob):
    batch = batch.astype(jnp.int32)
    z = z.astype(jnp.int32)

    starts = jnp.searchsorted(batch, jnp.arange(G + 1, dtype=jnp.int32),
                              side="left").astype(jnp.int32)
    r0 = jnp.arange(NB, dtype=jnp.int32) * RB
    g_lo = batch[r0]
    g_hi = batch[r0 + RB - 1]
    col_lo = starts[g_lo]
    col_hi = starts[g_hi + 1]
    tlo = col_lo // CT
    thi = (col_hi + CT - 1) // CT

    idx, cc, dmat = _nbr(pos, batch, tlo, thi)
    idxf = idx.reshape(N * K)
    cc3 = cc.reshape(N, K, 1)
    dflat = dmat.reshape(N * K, 1)

    rsub = jnp.arange(_NW, dtype=jnp.int32) * APW
    cmin = starts[batch[rsub]]
    cmax = starts[batch[rsub + APW - 1] + 1]
    w0s = jnp.clip(rsub - (WROWS - APW) // 2, 0, N - WROWS)
    fits = jnp.all((cmin >= w0s) & (cmax <= w0s + WROWS))

    emb_p = jnp.pad(emb, ((0, 0), (0, DP - H)))
    h = _sc_gather(emb_p, z, DP)

    def _agg_fast(ops):
        xl, wf, idxf = ops
        return _sc_agg(xl, wf, idxf)

    def _agg_slow(ops):
        xl, wf, idxf = ops
        xlp = jnp.pad(xl, ((0, 0), (0, DP - NF)))
        xg = _sc_gather(xlp, idxf, DP).reshape(N, K, DP)
        return _red(wf, xg)

    wfs = [_wker(dflat, cc3, mlp_w1[i], mlp_b1[i], mlp_w2[i], mlp_b2[i])
           for i in range(L)]
    for i in range(L):
        cf1p = jnp.pad(cf1_w[i], ((0, h.shape[1] - H), (0, 0)))
        xl = _lin(h, cf1p)
        agg = lax.cond(fits, _agg_fast, _agg_slow, (xl, wfs[i], idxf))
        h = _upd2(agg, h, cf2_w[i], cf2_b[i], il_w[i], il_b[i])
    return _readout(h, batch, hw1, hb1, hw2, hb2, ow, ob)
